# baseline (speedup 1.0000x reference)
import functools

import jax
import jax.numpy as jnp
from jax import lax
from jax.experimental import pallas as pl
from jax.experimental.pallas import tpu as pltpu
from jax.experimental.pallas import tpu_sc as plsc

N_NODES = 10000
N_EDGES = 320000
LATENT = 64

_NC = 2
_NS = 16
_NW = _NC * _NS
_EPW = N_EDGES // _NW
_CHUNK = 80
_NCHUNK = _EPW // _CHUNK
_NPT = 624

_EDGE_TILE = 2000
_NODE_TILE = 2000


def _dot(x, w):
    return jnp.dot(x, w, preferred_element_type=jnp.float32)


def _ln(x, scale, bias):
    mu = jnp.mean(x, axis=-1, keepdims=True)
    var = jnp.mean((x - mu) ** 2, axis=-1, keepdims=True)
    return (x - mu) * lax.rsqrt(var + 1e-5) * scale + bias


def _row_spec(tile, width):
    return pl.BlockSpec((tile, width), lambda i: (i, 0))


def _full_spec(shape):
    return pl.BlockSpec(shape, lambda i: (0, 0))


def _enc_node_kernel(nf, w0, b0, w1, b1, w2, b2, lns, lnb, wa, wb):
    def body(nf_r, w0_r, b0_r, w1_r, b1_r, w2_r, b2_r, s_r, t_r, wa_r, wb_r,
             nl_r, a_r, b_r):
        x = jnp.maximum(_dot(nf_r[...], w0_r[...]) + b0_r[...], 0.0)
        x = jnp.maximum(_dot(x, w1_r[...]) + b1_r[...], 0.0)
        x = _dot(x, w2_r[...]) + b2_r[...]
        nl = _ln(x, s_r[...], t_r[...])
        nl_r[...] = nl
        a_r[...] = _dot(nl, wa_r[...]).astype(jnp.bfloat16)
        b_r[...] = _dot(nl, wb_r[...]).astype(jnp.bfloat16)

    t = _NODE_TILE
    out = jax.ShapeDtypeStruct((N_NODES, LATENT), jnp.float32)
    outh = jax.ShapeDtypeStruct((N_NODES, LATENT), jnp.bfloat16)
    return pl.pallas_call(
        body,
        grid=(N_NODES // t,),
        in_specs=[_row_spec(t, nf.shape[1]),
                  _full_spec(w0.shape), _full_spec((1, LATENT)),
                  _full_spec(w1.shape), _full_spec((1, LATENT)),
                  _full_spec(w2.shape), _full_spec((1, LATENT)),
                  _full_spec((1, LATENT)), _full_spec((1, LATENT)),
                  _full_spec(wa.shape), _full_spec(wb.shape)],
        out_specs=[_row_spec(t, LATENT)] * 3,
        out_shape=[out, outh, outh],
    )(nf, w0, b0, w1, b1, w2, b2, lns, lnb, wa, wb)


def _enc_edge_kernel(ef, w0, b0, w1, b1, w2, b2, lns, lnb):
    def body(ef_r, w0_r, b0_r, w1_r, b1_r, w2_r, b2_r, s_r, t_r, el_r):
        x = jnp.maximum(_dot(ef_r[...], w0_r[...]) + b0_r[...], 0.0)
        x = jnp.maximum(_dot(x, w1_r[...]) + b1_r[...], 0.0)
        x = _dot(x, w2_r[...]) + b2_r[...]
        el_r[...] = _ln(x, s_r[...], t_r[...])

    t = _EDGE_TILE
    return pl.pallas_call(
        body,
        grid=(N_EDGES // t,),
        in_specs=[_row_spec(t, ef.shape[1]),
                  _full_spec(w0.shape), _full_spec((1, LATENT)),
                  _full_spec(w1.shape), _full_spec((1, LATENT)),
                  _full_spec(w2.shape), _full_spec((1, LATENT)),
                  _full_spec((1, LATENT)), _full_spec((1, LATENT))],
        out_specs=_row_spec(t, LATENT),
        out_shape=jax.ShapeDtypeStruct((N_EDGES, LATENT), jnp.float32),
    )(ef, w0, b0, w1, b1, w2, b2, lns, lnb)


def _edge_mlp_kernel(g1, g2, el, w1c, b1, w2, b2, w3, b3, lns, lnb):
    def body(g1_r, g2_r, el_r, w1c_r, b1_r, w2_r, b2_r, w3_r, b3_r, s_r, t_r,
             ne_r, eo_r):
        elv = el_r[...]
        g = g1_r[...].astype(jnp.float32) + g2_r[...].astype(jnp.float32)
        x = g + _dot(elv, w1c_r[...]) + b1_r[...]
        x = jnp.maximum(x, 0.0)
        x = jnp.maximum(_dot(x, w2_r[...]) + b2_r[...], 0.0)
        x = _dot(x, w3_r[...]) + b3_r[...]
        ne = _ln(x, s_r[...], t_r[...])
        ne_r[...] = ne
        eo_r[...] = elv + ne

    t = _EDGE_TILE
    out = jax.ShapeDtypeStruct((N_EDGES, LATENT), jnp.float32)
    return pl.pallas_call(
        body,
        grid=(N_EDGES // t,),
        in_specs=[_row_spec(t, LATENT)] * 3 +
                 [_full_spec((LATENT, LATENT)), _full_spec((1, LATENT)),
                  _full_spec((LATENT, LATENT)), _full_spec((1, LATENT)),
                  _full_spec((LATENT, LATENT)), _full_spec((1, LATENT)),
                  _full_spec((1, LATENT)), _full_spec((1, LATENT))],
        out_specs=[_row_spec(t, LATENT)] * 2,
        out_shape=[out, out],
    )(g1, g2, el, w1c, b1, w2, b2, w3, b3, lns, lnb)


def _node_mlp_kernel(nl, aggs, v1n, v1a, c1, v2, c2, v3, c3, lns, lnb, wa, wb):
    def body(nl_r, a0_r, a1_r, v1n_r, v1a_r, c1_r, v2_r, c2_r, v3_r, c3_r,
             s_r, t_r, wa_r, wb_r, no_r, pa_r, pb_r):
        nlv = nl_r[...]
        agg = a0_r[...][0] + a1_r[...][0]
        x = jnp.maximum(_dot(nlv, v1n_r[...]) + _dot(agg, v1a_r[...]) + c1_r[...], 0.0)
        x = jnp.maximum(_dot(x, v2_r[...]) + c2_r[...], 0.0)
        x = _dot(x, v3_r[...]) + c3_r[...]
        nn = nlv + _ln(x, s_r[...], t_r[...])
        no_r[...] = nn
        pa_r[...] = _dot(nn, wa_r[...]).astype(jnp.bfloat16)
        pb_r[...] = _dot(nn, wb_r[...]).astype(jnp.bfloat16)

    t = _NODE_TILE
    out = jax.ShapeDtypeStruct((N_NODES, LATENT), jnp.float32)
    outh = jax.ShapeDtypeStruct((N_NODES, LATENT), jnp.bfloat16)
    return pl.pallas_call(
        body,
        grid=(N_NODES // t,),
        in_specs=[_row_spec(t, LATENT),
                  pl.BlockSpec((1, t, LATENT), lambda i: (0, i, 0)),
                  pl.BlockSpec((1, t, LATENT), lambda i: (1, i, 0)),
                  _full_spec((LATENT, LATENT)), _full_spec((LATENT, LATENT)),
                  _full_spec((1, LATENT)),
                  _full_spec((LATENT, LATENT)), _full_spec((1, LATENT)),
                  _full_spec((LATENT, LATENT)), _full_spec((1, LATENT)),
                  _full_spec((1, LATENT)), _full_spec((1, LATENT)),
                  _full_spec((LATENT, LATENT)), _full_spec((LATENT, LATENT))],
        out_specs=[_row_spec(t, LATENT)] * 3,
        out_shape=[out, outh, outh],
    )(nl, aggs, aggs, v1n, v1a, c1, v2, c2, v3, c3, lns, lnb, wa, wb)


def _decoder_kernel(a_dec, d1, w2, d2, w3, d3):
    def body(a_r, d1_r, w2_r, d2_r, w3_r, d3_r, o_r):
        x = jnp.maximum(a_r[...] + d1_r[...], 0.0)
        x = jnp.maximum(_dot(x, w2_r[...]) + d2_r[...], 0.0)
        o_r[...] = _dot(x, w3_r[...]) + d3_r[...]

    t = _NODE_TILE
    k = w3.shape[1]
    return pl.pallas_call(
        body,
        grid=(N_NODES // t,),
        in_specs=[_row_spec(t, LATENT), _full_spec((1, LATENT)),
                  _full_spec((LATENT, LATENT)), _full_spec((1, LATENT)),
                  _full_spec((LATENT, k)), _full_spec((1, k))],
        out_specs=_row_spec(t, k),
        out_shape=jax.ShapeDtypeStruct((N_NODES, k), jnp.float32),
    )(a_dec, d1, w2, d2, w3, d3)


def _sc_gather(a_tab, b_tab, s3, r3):
    mesh = plsc.VectorSubcoreMesh(core_axis_name="c", subcore_axis_name="s")
    out = jax.ShapeDtypeStruct((N_EDGES, LATENT), jnp.bfloat16)

    @functools.partial(
        pl.kernel, mesh=mesh,
        out_type=[out, out],
        compiler_params=pltpu.CompilerParams(use_tc_tiling_on_sc=False),
        scratch_types=[
            pltpu.VMEM((_NCHUNK, _CHUNK), jnp.int32),
            pltpu.VMEM((_NCHUNK, _CHUNK), jnp.int32),
            pltpu.VMEM((_CHUNK, LATENT), jnp.bfloat16),
            pltpu.VMEM((_CHUNK, LATENT), jnp.bfloat16),
            pltpu.SemaphoreType.DMA,
            pltpu.SemaphoreType.DMA,
        ],
    )
    def k(a_hbm, b_hbm, s_hbm, r_hbm, g1_hbm, g2_hbm,
          sidx, ridx, bufa, bufb, sema, semb):
        wid = lax.axis_index("c") * _NS + lax.axis_index("s")
        pltpu.sync_copy(s_hbm.at[wid], sidx)
        pltpu.sync_copy(r_hbm.at[wid], ridx)
        base = wid * _EPW

        def body(j, carry):
            ca = pltpu.async_copy(a_hbm.at[sidx.at[j]], bufa, sema)
            cb = pltpu.async_copy(b_hbm.at[ridx.at[j]], bufb, semb)
            ca.wait()
            pltpu.sync_copy(bufa, g1_hbm.at[pl.ds(base + j * _CHUNK, _CHUNK)])
            cb.wait()
            pltpu.sync_copy(bufb, g2_hbm.at[pl.ds(base + j * _CHUNK, _CHUNK)])
            return carry

        lax.fori_loop(0, _NCHUNK, body, 0)

    return k(a_tab, b_tab, s3, r3)


def _sc_scatter(new_e, r3, zeros):
    mesh = plsc.VectorSubcoreMesh(core_axis_name="c", subcore_axis_name="s")

    @functools.partial(
        pl.kernel, mesh=mesh,
        out_type=jax.ShapeDtypeStruct((_NC, N_NODES, LATENT), jnp.float32),
        compiler_params=pltpu.CompilerParams(use_tc_tiling_on_sc=False),
        scratch_types=[
            pltpu.VMEM((_NCHUNK, _CHUNK), jnp.int32),
            pltpu.VMEM((_CHUNK, LATENT), jnp.float32),
            pltpu.VMEM_SHARED((N_NODES, LATENT), jnp.float32),
        ],
    )
    def k(ne_hbm, r_hbm, z_hbm, out_hbm, ridx, buf, agg):
        cid = lax.axis_index("c")
        sid = lax.axis_index("s")
        wid = cid * _NS + sid

        @pl.when(sid == 0)
        def _():
            pltpu.sync_copy(z_hbm, agg)

        plsc.subcore_barrier()
        pltpu.sync_copy(r_hbm.at[wid], ridx)
        base = wid * _EPW

        def body(j, carry):
            pltpu.sync_copy(ne_hbm.at[pl.ds(base + j * _CHUNK, _CHUNK)], buf)
            pltpu.sync_copy(buf, agg.at[ridx.at[j]], add=True)
            return carry

        lax.fori_loop(0, _NCHUNK, body, 0)
        plsc.subcore_barrier()
        pltpu.sync_copy(agg.at[pl.ds(sid * _NPT, _NPT)],
                        out_hbm.at[cid, pl.ds(sid * _NPT, _NPT)])

        @pl.when(sid == 0)
        def _tail():
            rem = _NS * _NPT
            pltpu.sync_copy(agg.at[pl.ds(rem, N_NODES - rem)],
                            out_hbm.at[cid, pl.ds(rem, N_NODES - rem)])

    return k(new_e, r3, zeros)


def _mlp_params(p):
    (w0, b0), (w1, b1), (w2, b2) = p["layers"]
    lns, lnb = p["ln"]
    return (w0, b0.reshape(1, -1), w1, b1.reshape(1, -1), w2,
            b2.reshape(1, -1), lns.reshape(1, -1), lnb.reshape(1, -1))


def _edge_l1_split(blk):
    w1 = blk["edge"]["layers"][0][0]
    return w1[:LATENT], w1[LATENT:2 * LATENT], w1[2 * LATENT:]


def kernel(node_features, edge_features, params, senders, receivers):
    nf = node_features[0]
    ef = edge_features[0]
    s3 = senders[0].reshape(_NW, _NCHUNK, _CHUNK)
    r3 = receivers[0].reshape(_NW, _NCHUNK, _CHUNK)
    zeros = jnp.zeros((N_NODES, LATENT), jnp.float32)
    blocks = params["blocks"]

    wa, wb, _ = _edge_l1_split(blocks[0])
    en = _mlp_params(params["enc_node"])
    nl, a_tab, b_tab = _enc_node_kernel(nf, *en, wa, wb)
    el = _enc_edge_kernel(ef, *_mlp_params(params["enc_edge"]))

    for i in range(len(blocks)):
        blk = blocks[i]
        _, _, w1c = _edge_l1_split(blk)
        ew = _mlp_params(blk["edge"])
        g1, g2 = _sc_gather(a_tab, b_tab, s3, r3)
        ne, el = _edge_mlp_kernel(g1, g2, el, w1c, ew[1], ew[2], ew[3],
                                  ew[4], ew[5], ew[6], ew[7])
        aggs = _sc_scatter(ne, r3, zeros)
        if i + 1 < len(blocks):
            wa, wb, _ = _edge_l1_split(blocks[i + 1])
        else:
            wa = params["dec"]["layers"][0][0]
            wb = wa
        nv = blk["node"]["layers"][0][0]
        nw = _mlp_params(blk["node"])
        nl, a_tab, b_tab = _node_mlp_kernel(
            nl, aggs, nv[:LATENT], nv[LATENT:], nw[1], nw[2], nw[3],
            nw[4], nw[5], nw[6], nw[7], wa, wb)

    dec = params["dec"]["layers"]
    out = _decoder_kernel(a_tab, dec[0][1].reshape(1, -1),
                          dec[1][0], dec[1][1].reshape(1, -1),
                          dec[2][0], dec[2][1].reshape(1, -1))
    return out.reshape(1, N_NODES, -1)


# speedup vs baseline: 1.1078x; 1.1078x over previous
import functools

import jax
import jax.numpy as jnp
from jax import lax
from jax.experimental import pallas as pl
from jax.experimental.pallas import tpu as pltpu
from jax.experimental.pallas import tpu_sc as plsc

N_NODES = 10000
N_EDGES = 320000
LATENT = 64

_NC = 2
_NS = 16
_NW = _NC * _NS
_EPW = N_EDGES // _NW
_CHUNK = 80
_NCHUNK = _EPW // _CHUNK
_NPT = 624

_NH = 2
_EH = N_EDGES // _NH
_EPW_H = _EH // _NW
_CHUNK_H = 40
_NCHUNK_H = _EPW_H // _CHUNK_H

_EDGE_TILE = 2000
_NODE_TILE = 2000


def _dot(x, w):
    return jnp.dot(x, w, preferred_element_type=jnp.float32)


def _ln(x, scale, bias):
    mu = jnp.mean(x, axis=-1, keepdims=True)
    var = jnp.mean((x - mu) ** 2, axis=-1, keepdims=True)
    return (x - mu) * lax.rsqrt(var + 1e-5) * scale + bias


def _row_spec(tile, width):
    return pl.BlockSpec((tile, width), lambda i: (i, 0))


def _full_spec(shape):
    return pl.BlockSpec(shape, lambda i: (0, 0))


def _enc_node_kernel(nf, w0, b0, w1, b1, w2, b2, lns, lnb, wa, wb):
    def body(nf_r, w0_r, b0_r, w1_r, b1_r, w2_r, b2_r, s_r, t_r, wa_r, wb_r,
             nl_r, a_r, b_r):
        x = jnp.maximum(_dot(nf_r[...], w0_r[...]) + b0_r[...], 0.0)
        x = jnp.maximum(_dot(x, w1_r[...]) + b1_r[...], 0.0)
        x = _dot(x, w2_r[...]) + b2_r[...]
        nl = _ln(x, s_r[...], t_r[...])
        nl_r[...] = nl
        a_r[...] = _dot(nl, wa_r[...])
        b_r[...] = _dot(nl, wb_r[...])

    t = _NODE_TILE
    out = jax.ShapeDtypeStruct((N_NODES, LATENT), jnp.float32)
    outh = out
    return pl.pallas_call(
        body,
        grid=(N_NODES // t,),
        in_specs=[_row_spec(t, nf.shape[1]),
                  _full_spec(w0.shape), _full_spec((1, LATENT)),
                  _full_spec(w1.shape), _full_spec((1, LATENT)),
                  _full_spec(w2.shape), _full_spec((1, LATENT)),
                  _full_spec((1, LATENT)), _full_spec((1, LATENT)),
                  _full_spec(wa.shape), _full_spec(wb.shape)],
        out_specs=[_row_spec(t, LATENT)] * 3,
        out_shape=[out, outh, outh],
    )(nf, w0, b0, w1, b1, w2, b2, lns, lnb, wa, wb)


def _enc_edge_kernel(ef, w0, b0, w1, b1, w2, b2, lns, lnb):
    def body(ef_r, w0_r, b0_r, w1_r, b1_r, w2_r, b2_r, s_r, t_r, el_r):
        x = jnp.maximum(_dot(ef_r[...], w0_r[...]) + b0_r[...], 0.0)
        x = jnp.maximum(_dot(x, w1_r[...]) + b1_r[...], 0.0)
        x = _dot(x, w2_r[...]) + b2_r[...]
        el_r[...] = _ln(x, s_r[...], t_r[...])

    t = _EDGE_TILE
    n = ef.shape[0]
    return pl.pallas_call(
        body,
        grid=(n // t,),
        in_specs=[_row_spec(t, ef.shape[1]),
                  _full_spec(w0.shape), _full_spec((1, LATENT)),
                  _full_spec(w1.shape), _full_spec((1, LATENT)),
                  _full_spec(w2.shape), _full_spec((1, LATENT)),
                  _full_spec((1, LATENT)), _full_spec((1, LATENT))],
        out_specs=_row_spec(t, LATENT),
        out_shape=jax.ShapeDtypeStruct((n, LATENT), jnp.float32),
    )(ef, w0, b0, w1, b1, w2, b2, lns, lnb)


def _edge_mlp_kernel(g1, g2, el, w1c, b1, w2, b2, w3, b3, lns, lnb):
    def body(g1_r, g2_r, el_r, w1c_r, b1_r, w2_r, b2_r, w3_r, b3_r, s_r, t_r,
             ne_r, eo_r):
        elv = el_r[...]
        x = g1_r[...] + g2_r[...] + _dot(elv, w1c_r[...]) + b1_r[...]
        x = jnp.maximum(x, 0.0)
        x = jnp.maximum(_dot(x, w2_r[...]) + b2_r[...], 0.0)
        x = _dot(x, w3_r[...]) + b3_r[...]
        ne = _ln(x, s_r[...], t_r[...])
        ne_r[...] = ne
        eo_r[...] = elv + ne

    t = _EDGE_TILE
    n = g1.shape[0]
    out = jax.ShapeDtypeStruct((n, LATENT), jnp.float32)
    return pl.pallas_call(
        body,
        grid=(n // t,),
        in_specs=[_row_spec(t, LATENT)] * 3 +
                 [_full_spec((LATENT, LATENT)), _full_spec((1, LATENT)),
                  _full_spec((LATENT, LATENT)), _full_spec((1, LATENT)),
                  _full_spec((LATENT, LATENT)), _full_spec((1, LATENT)),
                  _full_spec((1, LATENT)), _full_spec((1, LATENT))],
        out_specs=[_row_spec(t, LATENT)] * 2,
        out_shape=[out, out],
    )(g1, g2, el, w1c, b1, w2, b2, w3, b3, lns, lnb)


def _node_mlp_kernel(nl, aggs_a, aggs_b, v1n, v1a, c1, v2, c2, v3, c3,
                     lns, lnb, wa, wb):
    def body(nl_r, a0_r, a1_r, b0_r, b1_r, v1n_r, v1a_r, c1_r, v2_r, c2_r,
             v3_r, c3_r, s_r, t_r, wa_r, wb_r, no_r, pa_r, pb_r):
        nlv = nl_r[...]
        agg = (a0_r[...][0] + a1_r[...][0]) + (b0_r[...][0] + b1_r[...][0])
        x = jnp.maximum(_dot(nlv, v1n_r[...]) + _dot(agg, v1a_r[...]) + c1_r[...], 0.0)
        x = jnp.maximum(_dot(x, v2_r[...]) + c2_r[...], 0.0)
        x = _dot(x, v3_r[...]) + c3_r[...]
        nn = nlv + _ln(x, s_r[...], t_r[...])
        no_r[...] = nn
        pa_r[...] = _dot(nn, wa_r[...])
        pb_r[...] = _dot(nn, wb_r[...])

    t = _NODE_TILE
    out = jax.ShapeDtypeStruct((N_NODES, LATENT), jnp.float32)
    outh = out
    return pl.pallas_call(
        body,
        grid=(N_NODES // t,),
        in_specs=[_row_spec(t, LATENT),
                  pl.BlockSpec((1, t, LATENT), lambda i: (0, i, 0)),
                  pl.BlockSpec((1, t, LATENT), lambda i: (1, i, 0)),
                  pl.BlockSpec((1, t, LATENT), lambda i: (0, i, 0)),
                  pl.BlockSpec((1, t, LATENT), lambda i: (1, i, 0)),
                  _full_spec((LATENT, LATENT)), _full_spec((LATENT, LATENT)),
                  _full_spec((1, LATENT)),
                  _full_spec((LATENT, LATENT)), _full_spec((1, LATENT)),
                  _full_spec((LATENT, LATENT)), _full_spec((1, LATENT)),
                  _full_spec((1, LATENT)), _full_spec((1, LATENT)),
                  _full_spec((LATENT, LATENT)), _full_spec((LATENT, LATENT))],
        out_specs=[_row_spec(t, LATENT)] * 3,
        out_shape=[out, outh, outh],
    )(nl, aggs_a, aggs_a, aggs_b, aggs_b, v1n, v1a, c1, v2, c2, v3, c3,
      lns, lnb, wa, wb)


def _decoder_kernel(a_dec, d1, w2, d2, w3, d3):
    def body(a_r, d1_r, w2_r, d2_r, w3_r, d3_r, o_r):
        x = jnp.maximum(a_r[...] + d1_r[...], 0.0)
        x = jnp.maximum(_dot(x, w2_r[...]) + d2_r[...], 0.0)
        o_r[...] = _dot(x, w3_r[...]) + d3_r[...]

    t = _NODE_TILE
    k = w3.shape[1]
    return pl.pallas_call(
        body,
        grid=(N_NODES // t,),
        in_specs=[_row_spec(t, LATENT), _full_spec((1, LATENT)),
                  _full_spec((LATENT, LATENT)), _full_spec((1, LATENT)),
                  _full_spec((LATENT, k)), _full_spec((1, k))],
        out_specs=_row_spec(t, k),
        out_shape=jax.ShapeDtypeStruct((N_NODES, k), jnp.float32),
    )(a_dec, d1, w2, d2, w3, d3)


def _sc_gather(a_tab, b_tab, s3, r3, n_edges, chunk, nchunk):
    epw = n_edges // _NW
    mesh = plsc.VectorSubcoreMesh(core_axis_name="c", subcore_axis_name="s")
    out = jax.ShapeDtypeStruct((n_edges, LATENT), jnp.float32)

    @functools.partial(
        pl.kernel, mesh=mesh,
        out_type=[out, out],
        compiler_params=pltpu.CompilerParams(use_tc_tiling_on_sc=False),
        scratch_types=[
            pltpu.VMEM((nchunk, chunk), jnp.int32),
            pltpu.VMEM((nchunk, chunk), jnp.int32),
            pltpu.VMEM((chunk, LATENT), jnp.float32),
            pltpu.VMEM((chunk, LATENT), jnp.float32),
            pltpu.SemaphoreType.DMA,
            pltpu.SemaphoreType.DMA,
        ],
    )
    def k(a_hbm, b_hbm, s_hbm, r_hbm, g1_hbm, g2_hbm,
          sidx, ridx, bufa, bufb, sema, semb):
        wid = lax.axis_index("c") * _NS + lax.axis_index("s")
        pltpu.sync_copy(s_hbm.at[wid], sidx)
        pltpu.sync_copy(r_hbm.at[wid], ridx)
        base = wid * epw

        def body(j, carry):
            ca = pltpu.async_copy(a_hbm.at[sidx.at[j]], bufa, sema)
            cb = pltpu.async_copy(b_hbm.at[ridx.at[j]], bufb, semb)
            ca.wait()
            pltpu.sync_copy(bufa, g1_hbm.at[pl.ds(base + j * chunk, chunk)])
            cb.wait()
            pltpu.sync_copy(bufb, g2_hbm.at[pl.ds(base + j * chunk, chunk)])
            return carry

        lax.fori_loop(0, nchunk, body, 0)

    return k(a_tab, b_tab, s3, r3)


def _sc_scatter(new_e, r3, zeros, n_edges, chunk, nchunk):
    epw = n_edges // _NW
    mesh = plsc.VectorSubcoreMesh(core_axis_name="c", subcore_axis_name="s")

    @functools.partial(
        pl.kernel, mesh=mesh,
        out_type=jax.ShapeDtypeStruct((_NC, N_NODES, LATENT), jnp.float32),
        compiler_params=pltpu.CompilerParams(use_tc_tiling_on_sc=False),
        scratch_types=[
            pltpu.VMEM((nchunk, chunk), jnp.int32),
            pltpu.VMEM((chunk, LATENT), jnp.float32),
            pltpu.VMEM_SHARED((N_NODES, LATENT), jnp.float32),
        ],
    )
    def k(ne_hbm, r_hbm, z_hbm, out_hbm, ridx, buf, agg):
        cid = lax.axis_index("c")
        sid = lax.axis_index("s")
        wid = cid * _NS + sid

        @pl.when(sid == 0)
        def _():
            pltpu.sync_copy(z_hbm, agg)

        plsc.subcore_barrier()
        pltpu.sync_copy(r_hbm.at[wid], ridx)
        base = wid * epw

        def body(j, carry):
            pltpu.sync_copy(ne_hbm.at[pl.ds(base + j * chunk, chunk)], buf)
            pltpu.sync_copy(buf, agg.at[ridx.at[j]], add=True)
            return carry

        lax.fori_loop(0, nchunk, body, 0)
        plsc.subcore_barrier()
        pltpu.sync_copy(agg.at[pl.ds(sid * _NPT, _NPT)],
                        out_hbm.at[cid, pl.ds(sid * _NPT, _NPT)])

        @pl.when(sid == 0)
        def _tail():
            rem = _NS * _NPT
            pltpu.sync_copy(agg.at[pl.ds(rem, N_NODES - rem)],
                            out_hbm.at[cid, pl.ds(rem, N_NODES - rem)])

    return k(new_e, r3, zeros)


def _mlp_params(p):
    (w0, b0), (w1, b1), (w2, b2) = p["layers"]
    lns, lnb = p["ln"]
    return (w0, b0.reshape(1, -1), w1, b1.reshape(1, -1), w2,
            b2.reshape(1, -1), lns.reshape(1, -1), lnb.reshape(1, -1))


def _edge_l1_split(blk):
    w1 = blk["edge"]["layers"][0][0]
    return w1[:LATENT], w1[LATENT:2 * LATENT], w1[2 * LATENT:]


def kernel(node_features, edge_features, params, senders, receivers):
    nf = node_features[0]
    ef = edge_features[0]
    s = senders[0]
    r = receivers[0]
    s3 = [s[h * _EH:(h + 1) * _EH].reshape(_NW, _NCHUNK_H, _CHUNK_H)
          for h in range(_NH)]
    r3 = [r[h * _EH:(h + 1) * _EH].reshape(_NW, _NCHUNK_H, _CHUNK_H)
          for h in range(_NH)]
    zeros = jnp.zeros((N_NODES, LATENT), jnp.float32)
    blocks = params["blocks"]

    wa, wb, _ = _edge_l1_split(blocks[0])
    en = _mlp_params(params["enc_node"])
    nl, a_tab, b_tab = _enc_node_kernel(nf, *en, wa, wb)
    ee = _mlp_params(params["enc_edge"])
    el = [_enc_edge_kernel(ef[h * _EH:(h + 1) * _EH], *ee)
          for h in range(_NH)]

    for i in range(len(blocks)):
        blk = blocks[i]
        _, _, w1c = _edge_l1_split(blk)
        ew = _mlp_params(blk["edge"])
        g = [_sc_gather(a_tab, b_tab, s3[h], r3[h], _EH, _CHUNK_H, _NCHUNK_H)
             for h in range(_NH)]
        aggs = []
        for h in range(_NH):
            ne, el[h] = _edge_mlp_kernel(g[h][0], g[h][1], el[h], w1c, ew[1],
                                         ew[2], ew[3], ew[4], ew[5], ew[6],
                                         ew[7])
            aggs.append(_sc_scatter(ne, r3[h], zeros, _EH, _CHUNK_H,
                                    _NCHUNK_H))
        if i + 1 < len(blocks):
            wa, wb, _ = _edge_l1_split(blocks[i + 1])
        else:
            wa = params["dec"]["layers"][0][0]
            wb = wa
        nv = blk["node"]["layers"][0][0]
        nw = _mlp_params(blk["node"])
        nl, a_tab, b_tab = _node_mlp_kernel(
            nl, aggs[0], aggs[1], nv[:LATENT], nv[LATENT:], nw[1], nw[2],
            nw[3], nw[4], nw[5], nw[6], nw[7], wa, wb)

    dec = params["dec"]["layers"]
    out = _decoder_kernel(a_tab, dec[0][1].reshape(1, -1),
                          dec[1][0], dec[1][1].reshape(1, -1),
                          dec[2][0], dec[2][1].reshape(1, -1))
    return out.reshape(1, N_NODES, -1)


# speedup vs baseline: 1.2485x; 1.1270x over previous
import functools

import jax
import jax.numpy as jnp
from jax import lax
from jax.experimental import pallas as pl
from jax.experimental.pallas import tpu as pltpu
from jax.experimental.pallas import tpu_sc as plsc

N_NODES = 10000
N_EDGES = 320000
LATENT = 64

_NC = 2
_NS = 16
_NW = _NC * _NS
_EPW = N_EDGES // _NW
_CHUNK = 80
_NCHUNK = _EPW // _CHUNK
_NPT = 624

_NH = 2
_EH = N_EDGES // _NH
_EPW_H = _EH // _NW
_CHUNK_H = 40
_NCHUNK_H = _EPW_H // _CHUNK_H

_KSLOT = 5
_GPI = 5
_CPI = _KSLOT * _GPI

_EDGE_TILE = 2000
_NODE_TILE = 2000


def _dot(x, w):
    return jnp.dot(x, w, preferred_element_type=jnp.float32)


def _ln(x, scale, bias):
    mu = jnp.mean(x, axis=-1, keepdims=True)
    var = jnp.mean((x - mu) ** 2, axis=-1, keepdims=True)
    return (x - mu) * lax.rsqrt(var + 1e-5) * scale + bias


def _row_spec(tile, width):
    return pl.BlockSpec((tile, width), lambda i: (i, 0))


def _full_spec(shape):
    return pl.BlockSpec(shape, lambda i: (0, 0))


def _enc_node_kernel(nf, w0, b0, w1, b1, w2, b2, lns, lnb, wa, wb):
    def body(nf_r, w0_r, b0_r, w1_r, b1_r, w2_r, b2_r, s_r, t_r, wa_r, wb_r,
             nl_r, a_r, b_r):
        x = jnp.maximum(_dot(nf_r[...], w0_r[...]) + b0_r[...], 0.0)
        x = jnp.maximum(_dot(x, w1_r[...]) + b1_r[...], 0.0)
        x = _dot(x, w2_r[...]) + b2_r[...]
        nl = _ln(x, s_r[...], t_r[...])
        nl_r[...] = nl
        a_r[...] = _dot(nl, wa_r[...])
        b_r[...] = _dot(nl, wb_r[...])

    t = _NODE_TILE
    out = jax.ShapeDtypeStruct((N_NODES, LATENT), jnp.float32)
    outh = out
    return pl.pallas_call(
        body,
        grid=(N_NODES // t,),
        in_specs=[_row_spec(t, nf.shape[1]),
                  _full_spec(w0.shape), _full_spec((1, LATENT)),
                  _full_spec(w1.shape), _full_spec((1, LATENT)),
                  _full_spec(w2.shape), _full_spec((1, LATENT)),
                  _full_spec((1, LATENT)), _full_spec((1, LATENT)),
                  _full_spec(wa.shape), _full_spec(wb.shape)],
        out_specs=[_row_spec(t, LATENT)] * 3,
        out_shape=[out, outh, outh],
    )(nf, w0, b0, w1, b1, w2, b2, lns, lnb, wa, wb)


def _enc_edge_kernel(ef, w0, b0, w1, b1, w2, b2, lns, lnb):
    def body(ef_r, w0_r, b0_r, w1_r, b1_r, w2_r, b2_r, s_r, t_r, el_r):
        x = jnp.maximum(_dot(ef_r[...], w0_r[...]) + b0_r[...], 0.0)
        x = jnp.maximum(_dot(x, w1_r[...]) + b1_r[...], 0.0)
        x = _dot(x, w2_r[...]) + b2_r[...]
        el_r[...] = _ln(x, s_r[...], t_r[...])

    t = _EDGE_TILE
    n = ef.shape[0]
    return pl.pallas_call(
        body,
        grid=(n // t,),
        in_specs=[_row_spec(t, ef.shape[1]),
                  _full_spec(w0.shape), _full_spec((1, LATENT)),
                  _full_spec(w1.shape), _full_spec((1, LATENT)),
                  _full_spec(w2.shape), _full_spec((1, LATENT)),
                  _full_spec((1, LATENT)), _full_spec((1, LATENT))],
        out_specs=_row_spec(t, LATENT),
        out_shape=jax.ShapeDtypeStruct((n, LATENT), jnp.float32),
    )(ef, w0, b0, w1, b1, w2, b2, lns, lnb)


def _edge_mlp_kernel(g1, g2, el, w1c, b1, w2, b2, w3, b3, lns, lnb):
    def body(g1_r, g2_r, el_r, w1c_r, b1_r, w2_r, b2_r, w3_r, b3_r, s_r, t_r,
             ne_r, eo_r):
        elv = el_r[...]
        x = g1_r[...] + g2_r[...] + _dot(elv, w1c_r[...]) + b1_r[...]
        x = jnp.maximum(x, 0.0)
        x = jnp.maximum(_dot(x, w2_r[...]) + b2_r[...], 0.0)
        x = _dot(x, w3_r[...]) + b3_r[...]
        ne = _ln(x, s_r[...], t_r[...])
        ne_r[...] = ne
        eo_r[...] = elv + ne

    t = _EDGE_TILE
    n = g1.shape[0]
    out = jax.ShapeDtypeStruct((n, LATENT), jnp.float32)
    return pl.pallas_call(
        body,
        grid=(n // t,),
        in_specs=[_row_spec(t, LATENT)] * 3 +
                 [_full_spec((LATENT, LATENT)), _full_spec((1, LATENT)),
                  _full_spec((LATENT, LATENT)), _full_spec((1, LATENT)),
                  _full_spec((LATENT, LATENT)), _full_spec((1, LATENT)),
                  _full_spec((1, LATENT)), _full_spec((1, LATENT))],
        out_specs=[_row_spec(t, LATENT)] * 2,
        out_shape=[out, out],
    )(g1, g2, el, w1c, b1, w2, b2, w3, b3, lns, lnb)


def _node_mlp_kernel(nl, aggs_a, aggs_b, v1n, v1a, c1, v2, c2, v3, c3,
                     lns, lnb, wa, wb):
    def body(nl_r, a0_r, a1_r, b0_r, b1_r, v1n_r, v1a_r, c1_r, v2_r, c2_r,
             v3_r, c3_r, s_r, t_r, wa_r, wb_r, no_r, pa_r, pb_r):
        nlv = nl_r[...]
        agg = (a0_r[...][0] + a1_r[...][0]) + (b0_r[...][0] + b1_r[...][0])
        x = jnp.maximum(_dot(nlv, v1n_r[...]) + _dot(agg, v1a_r[...]) + c1_r[...], 0.0)
        x = jnp.maximum(_dot(x, v2_r[...]) + c2_r[...], 0.0)
        x = _dot(x, v3_r[...]) + c3_r[...]
        nn = nlv + _ln(x, s_r[...], t_r[...])
        no_r[...] = nn
        pa_r[...] = _dot(nn, wa_r[...])
        pb_r[...] = _dot(nn, wb_r[...])

    t = _NODE_TILE
    out = jax.ShapeDtypeStruct((N_NODES, LATENT), jnp.float32)
    outh = out
    return pl.pallas_call(
        body,
        grid=(N_NODES // t,),
        in_specs=[_row_spec(t, LATENT),
                  pl.BlockSpec((1, t, LATENT), lambda i: (0, i, 0)),
                  pl.BlockSpec((1, t, LATENT), lambda i: (1, i, 0)),
                  pl.BlockSpec((1, t, LATENT), lambda i: (0, i, 0)),
                  pl.BlockSpec((1, t, LATENT), lambda i: (1, i, 0)),
                  _full_spec((LATENT, LATENT)), _full_spec((LATENT, LATENT)),
                  _full_spec((1, LATENT)),
                  _full_spec((LATENT, LATENT)), _full_spec((1, LATENT)),
                  _full_spec((LATENT, LATENT)), _full_spec((1, LATENT)),
                  _full_spec((1, LATENT)), _full_spec((1, LATENT)),
                  _full_spec((LATENT, LATENT)), _full_spec((LATENT, LATENT))],
        out_specs=[_row_spec(t, LATENT)] * 3,
        out_shape=[out, outh, outh],
    )(nl, aggs_a, aggs_a, aggs_b, aggs_b, v1n, v1a, c1, v2, c2, v3, c3,
      lns, lnb, wa, wb)


def _decoder_kernel(a_dec, d1, w2, d2, w3, d3):
    def body(a_r, d1_r, w2_r, d2_r, w3_r, d3_r, o_r):
        x = jnp.maximum(a_r[...] + d1_r[...], 0.0)
        x = jnp.maximum(_dot(x, w2_r[...]) + d2_r[...], 0.0)
        o_r[...] = _dot(x, w3_r[...]) + d3_r[...]

    t = _NODE_TILE
    k = w3.shape[1]
    return pl.pallas_call(
        body,
        grid=(N_NODES // t,),
        in_specs=[_row_spec(t, LATENT), _full_spec((1, LATENT)),
                  _full_spec((LATENT, LATENT)), _full_spec((1, LATENT)),
                  _full_spec((LATENT, k)), _full_spec((1, k))],
        out_specs=_row_spec(t, k),
        out_shape=jax.ShapeDtypeStruct((N_NODES, k), jnp.float32),
    )(a_dec, d1, w2, d2, w3, d3)


def _sc_gather(a_tab, b_tab, s3, r3, n_edges, chunk, nchunk):
    epw = n_edges // _NW
    mesh = plsc.VectorSubcoreMesh(core_axis_name="c", subcore_axis_name="s")
    out = jax.ShapeDtypeStruct((n_edges, LATENT), jnp.float32)

    @functools.partial(
        pl.kernel, mesh=mesh,
        out_type=[out, out],
        compiler_params=pltpu.CompilerParams(use_tc_tiling_on_sc=False),
        scratch_types=[
            pltpu.VMEM((nchunk, chunk), jnp.int32),
            pltpu.VMEM((nchunk, chunk), jnp.int32),
            pltpu.VMEM((_KSLOT, chunk, LATENT), jnp.float32),
            pltpu.VMEM((_KSLOT, chunk, LATENT), jnp.float32),
            pltpu.VMEM((_KSLOT, chunk, LATENT), jnp.float32),
            pltpu.VMEM((_KSLOT, chunk, LATENT), jnp.float32),
            pltpu.SemaphoreType.DMA,
            pltpu.SemaphoreType.DMA,
            pltpu.SemaphoreType.DMA,
            pltpu.SemaphoreType.DMA,
        ],
    )
    def k(a_hbm, b_hbm, s_hbm, r_hbm, g1_hbm, g2_hbm,
          sidx, ridx, bufa0, bufa1, bufb0, bufb1, sema, semb, sem_sa, sem_sb):
        wid = lax.axis_index("c") * _NS + lax.axis_index("s")
        pltpu.sync_copy(s_hbm.at[wid], sidx)
        pltpu.sync_copy(r_hbm.at[wid], ridx)
        base = wid * epw
        bufa = (bufa0, bufa1)
        bufb = (bufb0, bufb1)

        def body(it, carry):
            c0 = it * _CPI

            def fire_g(g, s):
                ds = []
                for b in range(_KSLOT):
                    j = c0 + g * _KSLOT + b
                    da = pltpu.async_copy(a_hbm.at[sidx.at[j]], bufa[s].at[b], sema)
                    db = pltpu.async_copy(b_hbm.at[ridx.at[j]], bufb[s].at[b], semb)
                    ds.append((da, db))
                return ds

            def fire_s(g, s):
                ds = []
                for b in range(_KSLOT):
                    j = c0 + g * _KSLOT + b
                    off = base + j * chunk
                    da = pltpu.async_copy(bufa[s].at[b],
                                          g1_hbm.at[pl.ds(off, chunk)], sem_sa)
                    db = pltpu.async_copy(bufb[s].at[b],
                                          g2_hbm.at[pl.ds(off, chunk)], sem_sb)
                    ds.append((da, db))
                return ds

            gath = fire_g(0, 0)
            stores_prev = []
            for g in range(_GPI):
                s = g % 2
                for da, db in stores_prev:
                    da.wait()
                    db.wait()
                nxt = fire_g(g + 1, 1 - s) if g + 1 < _GPI else []
                for da, db in gath:
                    da.wait()
                    db.wait()
                stores_prev = fire_s(g, s)
                gath = nxt
            for da, db in stores_prev:
                da.wait()
                db.wait()
            return carry

        lax.fori_loop(0, nchunk // _CPI, body, 0)

    return k(a_tab, b_tab, s3, r3)


def _sc_scatter(new_e, r3, zeros, n_edges, chunk, nchunk):
    epw = n_edges // _NW
    mesh = plsc.VectorSubcoreMesh(core_axis_name="c", subcore_axis_name="s")

    @functools.partial(
        pl.kernel, mesh=mesh,
        out_type=jax.ShapeDtypeStruct((_NC, N_NODES, LATENT), jnp.float32),
        compiler_params=pltpu.CompilerParams(use_tc_tiling_on_sc=False),
        scratch_types=[
            pltpu.VMEM((nchunk, chunk), jnp.int32),
            pltpu.VMEM((_KSLOT, chunk, LATENT), jnp.float32),
            pltpu.VMEM((_KSLOT, chunk, LATENT), jnp.float32),
            pltpu.VMEM_SHARED((N_NODES, LATENT), jnp.float32),
            pltpu.SemaphoreType.DMA,
            pltpu.SemaphoreType.DMA,
        ],
    )
    def k(ne_hbm, r_hbm, z_hbm, out_hbm, ridx, buf0, buf1, agg, sem_l, sem_a):
        cid = lax.axis_index("c")
        sid = lax.axis_index("s")
        wid = cid * _NS + sid

        @pl.when(sid == 0)
        def _():
            pltpu.sync_copy(z_hbm, agg)

        plsc.subcore_barrier()
        pltpu.sync_copy(r_hbm.at[wid], ridx)
        base = wid * epw
        buf = (buf0, buf1)

        def body(it, carry):
            c0 = it * _CPI

            def fire_l(g, s):
                ds = []
                for b in range(_KSLOT):
                    j = c0 + g * _KSLOT + b
                    ds.append(pltpu.async_copy(
                        ne_hbm.at[pl.ds(base + j * chunk, chunk)],
                        buf[s].at[b], sem_l))
                return ds

            def fire_a(g, s):
                ds = []
                for b in range(_KSLOT):
                    j = c0 + g * _KSLOT + b
                    ds.append(pltpu.async_copy(
                        buf[s].at[b], agg.at[ridx.at[j]], sem_a, add=True))
                return ds

            loads = fire_l(0, 0)
            adds_prev = []
            for g in range(_GPI):
                s = g % 2
                for d in adds_prev:
                    d.wait()
                nxt = fire_l(g + 1, 1 - s) if g + 1 < _GPI else []
                for d in loads:
                    d.wait()
                adds_prev = fire_a(g, s)
                loads = nxt
            for d in adds_prev:
                d.wait()
            return carry

        lax.fori_loop(0, nchunk // _CPI, body, 0)
        plsc.subcore_barrier()
        pltpu.sync_copy(agg.at[pl.ds(sid * _NPT, _NPT)],
                        out_hbm.at[cid, pl.ds(sid * _NPT, _NPT)])

        @pl.when(sid == 0)
        def _tail():
            rem = _NS * _NPT
            pltpu.sync_copy(agg.at[pl.ds(rem, N_NODES - rem)],
                            out_hbm.at[cid, pl.ds(rem, N_NODES - rem)])

    return k(new_e, r3, zeros)


def _mlp_params(p):
    (w0, b0), (w1, b1), (w2, b2) = p["layers"]
    lns, lnb = p["ln"]
    return (w0, b0.reshape(1, -1), w1, b1.reshape(1, -1), w2,
            b2.reshape(1, -1), lns.reshape(1, -1), lnb.reshape(1, -1))


def _edge_l1_split(blk):
    w1 = blk["edge"]["layers"][0][0]
    return w1[:LATENT], w1[LATENT:2 * LATENT], w1[2 * LATENT:]


def kernel(node_features, edge_features, params, senders, receivers):
    nf = node_features[0]
    ef = edge_features[0]
    s = senders[0]
    r = receivers[0]
    s3 = [s[h * _EH:(h + 1) * _EH].reshape(_NW, _NCHUNK_H, _CHUNK_H)
          for h in range(_NH)]
    r3 = [r[h * _EH:(h + 1) * _EH].reshape(_NW, _NCHUNK_H, _CHUNK_H)
          for h in range(_NH)]
    zeros = jnp.zeros((N_NODES, LATENT), jnp.float32)
    blocks = params["blocks"]

    wa, wb, _ = _edge_l1_split(blocks[0])
    en = _mlp_params(params["enc_node"])
    nl, a_tab, b_tab = _enc_node_kernel(nf, *en, wa, wb)
    ee = _mlp_params(params["enc_edge"])
    el = [_enc_edge_kernel(ef[h * _EH:(h + 1) * _EH], *ee)
          for h in range(_NH)]

    for i in range(len(blocks)):
        blk = blocks[i]
        _, _, w1c = _edge_l1_split(blk)
        ew = _mlp_params(blk["edge"])
        g = [_sc_gather(a_tab, b_tab, s3[h], r3[h], _EH, _CHUNK_H, _NCHUNK_H)
             for h in range(_NH)]
        aggs = []
        for h in range(_NH):
            ne, el[h] = _edge_mlp_kernel(g[h][0], g[h][1], el[h], w1c, ew[1],
                                         ew[2], ew[3], ew[4], ew[5], ew[6],
                                         ew[7])
            aggs.append(_sc_scatter(ne, r3[h], zeros, _EH, _CHUNK_H,
                                    _NCHUNK_H))
        if i + 1 < len(blocks):
            wa, wb, _ = _edge_l1_split(blocks[i + 1])
        else:
            wa = params["dec"]["layers"][0][0]
            wb = wa
        nv = blk["node"]["layers"][0][0]
        nw = _mlp_params(blk["node"])
        nl, a_tab, b_tab = _node_mlp_kernel(
            nl, aggs[0], aggs[1], nv[:LATENT], nv[LATENT:], nw[1], nw[2],
            nw[3], nw[4], nw[5], nw[6], nw[7], wa, wb)

    dec = params["dec"]["layers"]
    out = _decoder_kernel(a_tab, dec[0][1].reshape(1, -1),
                          dec[1][0], dec[1][1].reshape(1, -1),
                          dec[2][0], dec[2][1].reshape(1, -1))
    return out.reshape(1, N_NODES, -1)


# speedup vs baseline: 1.6532x; 1.3241x over previous
import functools

import jax
import jax.numpy as jnp
from jax import lax
from jax.experimental import pallas as pl
from jax.experimental.pallas import tpu as pltpu
from jax.experimental.pallas import tpu_sc as plsc

N_NODES = 10000
N_EDGES = 320000
LATENT = 64

_NC = 2
_NS = 16
_NW = _NC * _NS
_EPW = N_EDGES // _NW
_CHUNK = 80
_NCHUNK = _EPW // _CHUNK
_NPT = 624

_NH = 2
_EH = N_EDGES // _NH
_EPW_H = _EH // _NW
_CHUNK_H = 40
_NCHUNK_H = _EPW_H // _CHUNK_H

_KSLOT = 5
_GPI = 5
_CPI = _KSLOT * _GPI

_EDGE_TILE = 2000
_NODE_TILE = 2000


def _dot(x, w):
    return jnp.dot(x, w, preferred_element_type=jnp.float32)


def _ln(x, scale, bias):
    mu = jnp.mean(x, axis=-1, keepdims=True)
    var = jnp.mean((x - mu) ** 2, axis=-1, keepdims=True)
    return (x - mu) * lax.rsqrt(var + 1e-5) * scale + bias


def _row_spec(tile, width):
    return pl.BlockSpec((tile, width), lambda i: (i, 0))


def _full_spec(shape):
    return pl.BlockSpec(shape, lambda i: (0, 0))


def _enc_node_kernel(nf, w0, b0, w1, b1, w2, b2, lns, lnb, wa, wb):
    def body(nf_r, w0_r, b0_r, w1_r, b1_r, w2_r, b2_r, s_r, t_r, wa_r, wb_r,
             nl_r, t_out_r):
        x = jnp.maximum(_dot(nf_r[...], w0_r[...]) + b0_r[...], 0.0)
        x = jnp.maximum(_dot(x, w1_r[...]) + b1_r[...], 0.0)
        x = _dot(x, w2_r[...]) + b2_r[...]
        nl = _ln(x, s_r[...], t_r[...])
        nl_r[...] = nl
        t_out_r[...] = jnp.concatenate(
            [_dot(nl, wa_r[...]), _dot(nl, wb_r[...])], axis=-1)

    t = _NODE_TILE
    out = jax.ShapeDtypeStruct((N_NODES, LATENT), jnp.float32)
    outt = jax.ShapeDtypeStruct((N_NODES, 2 * LATENT), jnp.float32)
    return pl.pallas_call(
        body,
        grid=(N_NODES // t,),
        in_specs=[_row_spec(t, nf.shape[1]),
                  _full_spec(w0.shape), _full_spec((1, LATENT)),
                  _full_spec(w1.shape), _full_spec((1, LATENT)),
                  _full_spec(w2.shape), _full_spec((1, LATENT)),
                  _full_spec((1, LATENT)), _full_spec((1, LATENT)),
                  _full_spec(wa.shape), _full_spec(wb.shape)],
        out_specs=[_row_spec(t, LATENT), _row_spec(t, 2 * LATENT)],
        out_shape=[out, outt],
    )(nf, w0, b0, w1, b1, w2, b2, lns, lnb, wa, wb)


def _enc_edge_kernel(ef, w0, b0, w1, b1, w2, b2, lns, lnb):
    def body(ef_r, w0_r, b0_r, w1_r, b1_r, w2_r, b2_r, s_r, t_r, el_r):
        x = jnp.maximum(_dot(ef_r[...], w0_r[...]) + b0_r[...], 0.0)
        x = jnp.maximum(_dot(x, w1_r[...]) + b1_r[...], 0.0)
        x = _dot(x, w2_r[...]) + b2_r[...]
        el_r[...] = _ln(x, s_r[...], t_r[...])

    t = _EDGE_TILE
    n = ef.shape[0]
    return pl.pallas_call(
        body,
        grid=(n // t,),
        in_specs=[_row_spec(t, ef.shape[1]),
                  _full_spec(w0.shape), _full_spec((1, LATENT)),
                  _full_spec(w1.shape), _full_spec((1, LATENT)),
                  _full_spec(w2.shape), _full_spec((1, LATENT)),
                  _full_spec((1, LATENT)), _full_spec((1, LATENT))],
        out_specs=_row_spec(t, LATENT),
        out_shape=jax.ShapeDtypeStruct((n, LATENT), jnp.float32),
    )(ef, w0, b0, w1, b1, w2, b2, lns, lnb)


def _edge_mlp_kernel(gs, gr, el, w1c, b1, w2, b2, w3, b3, lns, lnb):
    def body(gs_r, gr_r, el_r, w1c_r, b1_r, w2_r, b2_r, w3_r, b3_r, s_r, t_r,
             ne_r, eo_r):
        elv = el_r[...]
        x = (gs_r[...][:, :LATENT] + gr_r[...][:, LATENT:]
             + _dot(elv, w1c_r[...]) + b1_r[...])
        x = jnp.maximum(x, 0.0)
        x = jnp.maximum(_dot(x, w2_r[...]) + b2_r[...], 0.0)
        x = _dot(x, w3_r[...]) + b3_r[...]
        ne = _ln(x, s_r[...], t_r[...])
        ne_r[...] = jnp.concatenate([ne, jnp.zeros_like(ne)], axis=-1)
        eo_r[...] = elv + ne

    t = _EDGE_TILE
    n = gs.shape[0]
    out = jax.ShapeDtypeStruct((n, LATENT), jnp.float32)
    outp = jax.ShapeDtypeStruct((n, 2 * LATENT), jnp.float32)
    return pl.pallas_call(
        body,
        grid=(n // t,),
        in_specs=[_row_spec(t, 2 * LATENT), _row_spec(t, 2 * LATENT),
                  _row_spec(t, LATENT),
                  _full_spec((LATENT, LATENT)), _full_spec((1, LATENT)),
                  _full_spec((LATENT, LATENT)), _full_spec((1, LATENT)),
                  _full_spec((LATENT, LATENT)), _full_spec((1, LATENT)),
                  _full_spec((1, LATENT)), _full_spec((1, LATENT))],
        out_specs=[_row_spec(t, 2 * LATENT), _row_spec(t, LATENT)],
        out_shape=[outp, out],
    )(gs, gr, el, w1c, b1, w2, b2, w3, b3, lns, lnb)


def _node_mlp_kernel(nl, aggs_a, aggs_b, v1n, v1a, c1, v2, c2, v3, c3,
                     lns, lnb, wa, wb):
    def body(nl_r, a0_r, a1_r, b0_r, b1_r, v1n_r, v1a_r, c1_r, v2_r, c2_r,
             v3_r, c3_r, s_r, t_r, wa_r, wb_r, no_r, t_out_r):
        nlv = nl_r[...]
        agg = ((a0_r[...][0] + a1_r[...][0])
               + (b0_r[...][0] + b1_r[...][0]))[:, :LATENT]
        x = jnp.maximum(_dot(nlv, v1n_r[...]) + _dot(agg, v1a_r[...]) + c1_r[...], 0.0)
        x = jnp.maximum(_dot(x, v2_r[...]) + c2_r[...], 0.0)
        x = _dot(x, v3_r[...]) + c3_r[...]
        nn = nlv + _ln(x, s_r[...], t_r[...])
        no_r[...] = nn
        t_out_r[...] = jnp.concatenate(
            [_dot(nn, wa_r[...]), _dot(nn, wb_r[...])], axis=-1)

    t = _NODE_TILE
    out = jax.ShapeDtypeStruct((N_NODES, LATENT), jnp.float32)
    outt = jax.ShapeDtypeStruct((N_NODES, 2 * LATENT), jnp.float32)
    return pl.pallas_call(
        body,
        grid=(N_NODES // t,),
        in_specs=[_row_spec(t, LATENT),
                  pl.BlockSpec((1, t, 2 * LATENT), lambda i: (0, i, 0)),
                  pl.BlockSpec((1, t, 2 * LATENT), lambda i: (1, i, 0)),
                  pl.BlockSpec((1, t, 2 * LATENT), lambda i: (0, i, 0)),
                  pl.BlockSpec((1, t, 2 * LATENT), lambda i: (1, i, 0)),
                  _full_spec((LATENT, LATENT)), _full_spec((LATENT, LATENT)),
                  _full_spec((1, LATENT)),
                  _full_spec((LATENT, LATENT)), _full_spec((1, LATENT)),
                  _full_spec((LATENT, LATENT)), _full_spec((1, LATENT)),
                  _full_spec((1, LATENT)), _full_spec((1, LATENT)),
                  _full_spec((LATENT, LATENT)), _full_spec((LATENT, LATENT))],
        out_specs=[_row_spec(t, LATENT), _row_spec(t, 2 * LATENT)],
        out_shape=[out, outt],
    )(nl, aggs_a, aggs_a, aggs_b, aggs_b, v1n, v1a, c1, v2, c2, v3, c3,
      lns, lnb, wa, wb)


def _decoder_kernel(a_dec, d1, w2, d2, w3, d3):
    def body(a_r, d1_r, w2_r, d2_r, w3_r, d3_r, o_r):
        x = jnp.maximum(a_r[...][:, :LATENT] + d1_r[...], 0.0)
        x = jnp.maximum(_dot(x, w2_r[...]) + d2_r[...], 0.0)
        o_r[...] = _dot(x, w3_r[...]) + d3_r[...]

    t = _NODE_TILE
    k = w3.shape[1]
    return pl.pallas_call(
        body,
        grid=(N_NODES // t,),
        in_specs=[_row_spec(t, 2 * LATENT), _full_spec((1, LATENT)),
                  _full_spec((LATENT, LATENT)), _full_spec((1, LATENT)),
                  _full_spec((LATENT, k)), _full_spec((1, k))],
        out_specs=_row_spec(t, k),
        out_shape=jax.ShapeDtypeStruct((N_NODES, k), jnp.float32),
    )(a_dec, d1, w2, d2, w3, d3)


def _sc_gather(t_tab, s3, r3, n_edges, chunk, nchunk):
    epw = n_edges // _NW
    mesh = plsc.VectorSubcoreMesh(core_axis_name="c", subcore_axis_name="s")
    out = jax.ShapeDtypeStruct((n_edges, 2 * LATENT), jnp.float32)
    ksl = 4
    gpi = 5
    cpi = ksl * gpi
    nit = nchunk // cpi

    @functools.partial(
        pl.kernel, mesh=mesh,
        out_type=[out, out],
        scratch_types=[
            pltpu.VMEM((nchunk, chunk), jnp.int32),
            pltpu.VMEM((nchunk, chunk), jnp.int32),
            pltpu.VMEM((ksl, chunk, 2 * LATENT), jnp.float32),
            pltpu.VMEM((ksl, chunk, 2 * LATENT), jnp.float32),
            pltpu.VMEM((ksl, chunk, 2 * LATENT), jnp.float32),
            pltpu.VMEM((ksl, chunk, 2 * LATENT), jnp.float32),
            pltpu.SemaphoreType.DMA,
            pltpu.SemaphoreType.DMA,
            pltpu.SemaphoreType.DMA,
            pltpu.SemaphoreType.DMA,
        ],
    )
    def k(t_hbm, s_hbm, r_hbm, gs_hbm, gr_hbm,
          sidx, ridx, bufa0, bufa1, bufb0, bufb1, sema, semb, sem_sa, sem_sb):
        wid = lax.axis_index("c") * _NS + lax.axis_index("s")
        pltpu.sync_copy(s_hbm.at[wid], sidx)
        pltpu.sync_copy(r_hbm.at[wid], ridx)
        base = wid * epw
        bufa = (bufa0, bufa1)
        bufb = (bufb0, bufb1)

        def body(it, carry):
            c0 = it * cpi

            def fire_g(g, s):
                ds = []
                for b in range(ksl):
                    j = c0 + g * ksl + b
                    da = pltpu.async_copy(t_hbm.at[sidx.at[j]], bufa[s].at[b], sema)
                    db = pltpu.async_copy(t_hbm.at[ridx.at[j]], bufb[s].at[b], semb)
                    ds.append((da, db))
                return ds

            def fire_s(g, s):
                ds = []
                for b in range(ksl):
                    j = c0 + g * ksl + b
                    off = base + j * chunk
                    da = pltpu.async_copy(bufa[s].at[b],
                                          gs_hbm.at[pl.ds(off, chunk)], sem_sa)
                    db = pltpu.async_copy(bufb[s].at[b],
                                          gr_hbm.at[pl.ds(off, chunk)], sem_sb)
                    ds.append((da, db))
                return ds

            gath = fire_g(0, 0)
            stores_prev = []
            for g in range(gpi):
                s = g % 2
                for da, db in stores_prev:
                    da.wait()
                    db.wait()
                nxt = fire_g(g + 1, 1 - s) if g + 1 < gpi else []
                for da, db in gath:
                    da.wait()
                    db.wait()
                stores_prev = fire_s(g, s)
                gath = nxt
            for da, db in stores_prev:
                da.wait()
                db.wait()
            return carry

        lax.fori_loop(0, nit, body, 0)

        for j in range(nit * cpi, nchunk):
            off = base + j * chunk
            ca = pltpu.async_copy(t_hbm.at[sidx.at[j]], bufa[0].at[0], sema)
            cb = pltpu.async_copy(t_hbm.at[ridx.at[j]], bufb[0].at[0], semb)
            ca.wait()
            cb.wait()
            pltpu.sync_copy(bufa[0].at[0], gs_hbm.at[pl.ds(off, chunk)])
            pltpu.sync_copy(bufb[0].at[0], gr_hbm.at[pl.ds(off, chunk)])

    return k(t_tab, s3, r3)


def _sc_scatter(new_e, r3, zeros, n_edges, chunk, nchunk):
    epw = n_edges // _NW
    mesh = plsc.VectorSubcoreMesh(core_axis_name="c", subcore_axis_name="s")

    ksl = 3
    gpi = 5
    cpi = ksl * gpi
    nit = nchunk // cpi

    @functools.partial(
        pl.kernel, mesh=mesh,
        out_type=jax.ShapeDtypeStruct((_NC, N_NODES, 2 * LATENT), jnp.float32),
        scratch_types=[
            pltpu.VMEM((nchunk, chunk), jnp.int32),
            pltpu.VMEM((ksl, chunk, 2 * LATENT), jnp.float32),
            pltpu.VMEM((ksl, chunk, 2 * LATENT), jnp.float32),
            pltpu.VMEM_SHARED((N_NODES, 2 * LATENT), jnp.float32),
            pltpu.SemaphoreType.DMA,
            pltpu.SemaphoreType.DMA,
        ],
    )
    def k(ne_hbm, r_hbm, z_hbm, out_hbm, ridx, buf0, buf1, agg, sem_l, sem_a):
        cid = lax.axis_index("c")
        sid = lax.axis_index("s")
        wid = cid * _NS + sid

        @pl.when(sid == 0)
        def _():
            pltpu.sync_copy(z_hbm, agg)

        plsc.subcore_barrier()
        pltpu.sync_copy(r_hbm.at[wid], ridx)
        base = wid * epw
        buf = (buf0, buf1)

        def body(it, carry):
            c0 = it * cpi

            def fire_l(g, s):
                ds = []
                for b in range(ksl):
                    j = c0 + g * ksl + b
                    ds.append(pltpu.async_copy(
                        ne_hbm.at[pl.ds(base + j * chunk, chunk)],
                        buf[s].at[b], sem_l))
                return ds

            def fire_a(g, s):
                ds = []
                for b in range(ksl):
                    j = c0 + g * ksl + b
                    ds.append(pltpu.async_copy(
                        buf[s].at[b], agg.at[ridx.at[j]], sem_a, add=True))
                return ds

            loads = fire_l(0, 0)
            adds_prev = []
            for g in range(gpi):
                s = g % 2
                for d in adds_prev:
                    d.wait()
                nxt = fire_l(g + 1, 1 - s) if g + 1 < gpi else []
                for d in loads:
                    d.wait()
                adds_prev = fire_a(g, s)
                loads = nxt
            for d in adds_prev:
                d.wait()
            return carry

        lax.fori_loop(0, nit, body, 0)

        for j in range(nit * cpi, nchunk):
            pltpu.sync_copy(ne_hbm.at[pl.ds(base + j * chunk, chunk)],
                            buf[0].at[0])
            pltpu.sync_copy(buf[0].at[0], agg.at[ridx.at[j]], add=True)
        plsc.subcore_barrier()
        pltpu.sync_copy(agg.at[pl.ds(sid * _NPT, _NPT)],
                        out_hbm.at[cid, pl.ds(sid * _NPT, _NPT)])

        @pl.when(sid == 0)
        def _tail():
            rem = _NS * _NPT
            pltpu.sync_copy(agg.at[pl.ds(rem, N_NODES - rem)],
                            out_hbm.at[cid, pl.ds(rem, N_NODES - rem)])

    return k(new_e, r3, zeros)


def _mlp_params(p):
    (w0, b0), (w1, b1), (w2, b2) = p["layers"]
    lns, lnb = p["ln"]
    return (w0, b0.reshape(1, -1), w1, b1.reshape(1, -1), w2,
            b2.reshape(1, -1), lns.reshape(1, -1), lnb.reshape(1, -1))


def _edge_l1_split(blk):
    w1 = blk["edge"]["layers"][0][0]
    return w1[:LATENT], w1[LATENT:2 * LATENT], w1[2 * LATENT:]


def kernel(node_features, edge_features, params, senders, receivers):
    nf = node_features[0]
    ef = edge_features[0]
    s = senders[0]
    r = receivers[0]
    s3 = [s[h * _EH:(h + 1) * _EH].reshape(_NW, _NCHUNK_H, _CHUNK_H)
          for h in range(_NH)]
    r3 = [r[h * _EH:(h + 1) * _EH].reshape(_NW, _NCHUNK_H, _CHUNK_H)
          for h in range(_NH)]
    zeros = jnp.zeros((N_NODES, 2 * LATENT), jnp.float32)
    blocks = params["blocks"]

    wa, wb, _ = _edge_l1_split(blocks[0])
    en = _mlp_params(params["enc_node"])
    nl, t_tab = _enc_node_kernel(nf, *en, wa, wb)
    ee = _mlp_params(params["enc_edge"])
    el = [_enc_edge_kernel(ef[h * _EH:(h + 1) * _EH], *ee)
          for h in range(_NH)]

    for i in range(len(blocks)):
        blk = blocks[i]
        _, _, w1c = _edge_l1_split(blk)
        ew = _mlp_params(blk["edge"])
        g = [_sc_gather(t_tab, s3[h], r3[h], _EH, _CHUNK_H, _NCHUNK_H)
             for h in range(_NH)]
        aggs = []
        for h in range(_NH):
            ne, el[h] = _edge_mlp_kernel(g[h][0], g[h][1], el[h], w1c, ew[1],
                                         ew[2], ew[3], ew[4], ew[5], ew[6],
                                         ew[7])
            aggs.append(_sc_scatter(ne, r3[h], zeros, _EH, _CHUNK_H,
                                    _NCHUNK_H))
        if i + 1 < len(blocks):
            wa, wb, _ = _edge_l1_split(blocks[i + 1])
        else:
            wa = params["dec"]["layers"][0][0]
            wb = wa
        nv = blk["node"]["layers"][0][0]
        nw = _mlp_params(blk["node"])
        nl, t_tab = _node_mlp_kernel(
            nl, aggs[0], aggs[1], nv[:LATENT], nv[LATENT:], nw[1], nw[2],
            nw[3], nw[4], nw[5], nw[6], nw[7], wa, wb)

    dec = params["dec"]["layers"]
    out = _decoder_kernel(t_tab, dec[0][1].reshape(1, -1),
                          dec[1][0], dec[1][1].reshape(1, -1),
                          dec[2][0], dec[2][1].reshape(1, -1))
    return out.reshape(1, N_NODES, -1)


# speedup vs baseline: 1.7148x; 1.0372x over previous
import functools

import jax
import jax.numpy as jnp
from jax import lax
from jax.experimental import pallas as pl
from jax.experimental.pallas import tpu as pltpu
from jax.experimental.pallas import tpu_sc as plsc

N_NODES = 10000
N_EDGES = 320000
LATENT = 64

_NC = 2
_NS = 16
_NW = _NC * _NS
_EPW = N_EDGES // _NW
_CHUNK = 80
_NCHUNK = _EPW // _CHUNK
_NPT = 624

_NH = 2
_EH = N_EDGES // _NH
_EPW_H = _EH // _NW
_CHUNK_H = 40
_NCHUNK_H = _EPW_H // _CHUNK_H

_KSLOT = 5
_GPI = 5
_CPI = _KSLOT * _GPI

_EDGE_TILE = 2000
_NODE_TILE = 2000


def _dot(x, w):
    return jnp.dot(x, w, preferred_element_type=jnp.float32)


def _ln(x, scale, bias):
    mu = jnp.mean(x, axis=-1, keepdims=True)
    var = jnp.mean((x - mu) ** 2, axis=-1, keepdims=True)
    return (x - mu) * lax.rsqrt(var + 1e-5) * scale + bias


def _row_spec(tile, width):
    return pl.BlockSpec((tile, width), lambda i: (i, 0))


def _full_spec(shape):
    return pl.BlockSpec(shape, lambda i: (0, 0))


def _enc_node_kernel(nf, w0, b0, w1, b1, w2, b2, lns, lnb, wa, wb):
    def body(nf_r, w0_r, b0_r, w1_r, b1_r, w2_r, b2_r, s_r, t_r, wa_r, wb_r,
             nl_r, t_out_r):
        x = jnp.maximum(_dot(nf_r[...], w0_r[...]) + b0_r[...], 0.0)
        x = jnp.maximum(_dot(x, w1_r[...]) + b1_r[...], 0.0)
        x = _dot(x, w2_r[...]) + b2_r[...]
        nl = _ln(x, s_r[...], t_r[...])
        nl_r[...] = nl
        t_out_r[...] = jnp.concatenate(
            [_dot(nl, wa_r[...]), _dot(nl, wb_r[...])], axis=-1)

    t = _NODE_TILE
    out = jax.ShapeDtypeStruct((N_NODES, LATENT), jnp.float32)
    outt = jax.ShapeDtypeStruct((N_NODES, 2 * LATENT), jnp.float32)
    return pl.pallas_call(
        body,
        grid=(N_NODES // t,),
        in_specs=[_row_spec(t, nf.shape[1]),
                  _full_spec(w0.shape), _full_spec((1, LATENT)),
                  _full_spec(w1.shape), _full_spec((1, LATENT)),
                  _full_spec(w2.shape), _full_spec((1, LATENT)),
                  _full_spec((1, LATENT)), _full_spec((1, LATENT)),
                  _full_spec(wa.shape), _full_spec(wb.shape)],
        out_specs=[_row_spec(t, LATENT), _row_spec(t, 2 * LATENT)],
        out_shape=[out, outt],
    )(nf, w0, b0, w1, b1, w2, b2, lns, lnb, wa, wb)


def _enc_edge_kernel(ef, w0, b0, w1, b1, w2, b2, lns, lnb):
    def body(ef_r, w0_r, b0_r, w1_r, b1_r, w2_r, b2_r, s_r, t_r, el_r):
        x = jnp.maximum(_dot(ef_r[...], w0_r[...]) + b0_r[...], 0.0)
        x = jnp.maximum(_dot(x, w1_r[...]) + b1_r[...], 0.0)
        x = _dot(x, w2_r[...]) + b2_r[...]
        el_r[...] = _ln(x, s_r[...], t_r[...])

    t = _EDGE_TILE
    n = ef.shape[0]
    return pl.pallas_call(
        body,
        grid=(n // t,),
        in_specs=[_row_spec(t, ef.shape[1]),
                  _full_spec(w0.shape), _full_spec((1, LATENT)),
                  _full_spec(w1.shape), _full_spec((1, LATENT)),
                  _full_spec(w2.shape), _full_spec((1, LATENT)),
                  _full_spec((1, LATENT)), _full_spec((1, LATENT))],
        out_specs=_row_spec(t, LATENT),
        out_shape=jax.ShapeDtypeStruct((n, LATENT), jnp.float32),
    )(ef, w0, b0, w1, b1, w2, b2, lns, lnb)


def _edge_mlp_kernel(gs, gr, el, w1c, b1, w2, b2, w3, b3, lns, lnb):
    def body(gs_r, gr_r, el_r, w1c_r, b1_r, w2_r, b2_r, w3_r, b3_r, s_r, t_r,
             ne_r, eo_r):
        elv = el_r[...]
        x = (gs_r[...][:, :LATENT] + gr_r[...][:, LATENT:]
             + _dot(elv, w1c_r[...]) + b1_r[...])
        x = jnp.maximum(x, 0.0)
        x = jnp.maximum(_dot(x, w2_r[...]) + b2_r[...], 0.0)
        x = _dot(x, w3_r[...]) + b3_r[...]
        ne = _ln(x, s_r[...], t_r[...])
        ne_r[...] = jnp.concatenate([ne, jnp.zeros_like(ne)], axis=-1)
        eo_r[...] = elv + ne

    t = _EDGE_TILE
    n = gs.shape[0]
    out = jax.ShapeDtypeStruct((n, LATENT), jnp.float32)
    outp = jax.ShapeDtypeStruct((n, 2 * LATENT), jnp.float32)
    return pl.pallas_call(
        body,
        grid=(n // t,),
        in_specs=[_row_spec(t, 2 * LATENT), _row_spec(t, 2 * LATENT),
                  _row_spec(t, LATENT),
                  _full_spec((LATENT, LATENT)), _full_spec((1, LATENT)),
                  _full_spec((LATENT, LATENT)), _full_spec((1, LATENT)),
                  _full_spec((LATENT, LATENT)), _full_spec((1, LATENT)),
                  _full_spec((1, LATENT)), _full_spec((1, LATENT))],
        out_specs=[_row_spec(t, 2 * LATENT), _row_spec(t, LATENT)],
        out_shape=[outp, out],
    )(gs, gr, el, w1c, b1, w2, b2, w3, b3, lns, lnb)


def _edge_enc_mlp_kernel(gs, gr, ef, enc, w1c, b1, w2, b2, w3, b3, lns, lnb):
    ew0, eb0, ew1, eb1, ew2, eb2, elns, elnb = enc

    def body(gs_r, gr_r, ef_r, ew0_r, eb0_r, ew1_r, eb1_r, ew2_r, eb2_r,
             es_r, et_r, w1c_r, b1_r, w2_r, b2_r, w3_r, b3_r, s_r, t_r,
             ne_r, eo_r):
        e = jnp.maximum(_dot(ef_r[...], ew0_r[...]) + eb0_r[...], 0.0)
        e = jnp.maximum(_dot(e, ew1_r[...]) + eb1_r[...], 0.0)
        e = _dot(e, ew2_r[...]) + eb2_r[...]
        elv = _ln(e, es_r[...], et_r[...])
        x = (gs_r[...][:, :LATENT] + gr_r[...][:, LATENT:]
             + _dot(elv, w1c_r[...]) + b1_r[...])
        x = jnp.maximum(x, 0.0)
        x = jnp.maximum(_dot(x, w2_r[...]) + b2_r[...], 0.0)
        x = _dot(x, w3_r[...]) + b3_r[...]
        ne = _ln(x, s_r[...], t_r[...])
        ne_r[...] = jnp.concatenate([ne, jnp.zeros_like(ne)], axis=-1)
        eo_r[...] = elv + ne

    t = _EDGE_TILE
    n = gs.shape[0]
    out = jax.ShapeDtypeStruct((n, LATENT), jnp.float32)
    outp = jax.ShapeDtypeStruct((n, 2 * LATENT), jnp.float32)
    return pl.pallas_call(
        body,
        grid=(n // t,),
        in_specs=[_row_spec(t, 2 * LATENT), _row_spec(t, 2 * LATENT),
                  _row_spec(t, ef.shape[1]),
                  _full_spec(ew0.shape), _full_spec((1, LATENT)),
                  _full_spec(ew1.shape), _full_spec((1, LATENT)),
                  _full_spec(ew2.shape), _full_spec((1, LATENT)),
                  _full_spec((1, LATENT)), _full_spec((1, LATENT)),
                  _full_spec((LATENT, LATENT)), _full_spec((1, LATENT)),
                  _full_spec((LATENT, LATENT)), _full_spec((1, LATENT)),
                  _full_spec((LATENT, LATENT)), _full_spec((1, LATENT)),
                  _full_spec((1, LATENT)), _full_spec((1, LATENT))],
        out_specs=[_row_spec(t, 2 * LATENT), _row_spec(t, LATENT)],
        out_shape=[outp, out],
    )(gs, gr, ef, ew0, eb0, ew1, eb1, ew2, eb2, elns, elnb,
      w1c, b1, w2, b2, w3, b3, lns, lnb)


def _node_mlp_kernel(nl, aggs_a, aggs_b, v1n, v1a, c1, v2, c2, v3, c3,
                     lns, lnb, wa, wb):
    def body(nl_r, a0_r, a1_r, b0_r, b1_r, v1n_r, v1a_r, c1_r, v2_r, c2_r,
             v3_r, c3_r, s_r, t_r, wa_r, wb_r, no_r, t_out_r):
        nlv = nl_r[...]
        agg = ((a0_r[...][0] + a1_r[...][0])
               + (b0_r[...][0] + b1_r[...][0]))[:, :LATENT]
        x = jnp.maximum(_dot(nlv, v1n_r[...]) + _dot(agg, v1a_r[...]) + c1_r[...], 0.0)
        x = jnp.maximum(_dot(x, v2_r[...]) + c2_r[...], 0.0)
        x = _dot(x, v3_r[...]) + c3_r[...]
        nn = nlv + _ln(x, s_r[...], t_r[...])
        no_r[...] = nn
        t_out_r[...] = jnp.concatenate(
            [_dot(nn, wa_r[...]), _dot(nn, wb_r[...])], axis=-1)

    t = _NODE_TILE
    out = jax.ShapeDtypeStruct((N_NODES, LATENT), jnp.float32)
    outt = jax.ShapeDtypeStruct((N_NODES, 2 * LATENT), jnp.float32)
    return pl.pallas_call(
        body,
        grid=(N_NODES // t,),
        in_specs=[_row_spec(t, LATENT),
                  pl.BlockSpec((1, t, 2 * LATENT), lambda i: (0, i, 0)),
                  pl.BlockSpec((1, t, 2 * LATENT), lambda i: (1, i, 0)),
                  pl.BlockSpec((1, t, 2 * LATENT), lambda i: (0, i, 0)),
                  pl.BlockSpec((1, t, 2 * LATENT), lambda i: (1, i, 0)),
                  _full_spec((LATENT, LATENT)), _full_spec((LATENT, LATENT)),
                  _full_spec((1, LATENT)),
                  _full_spec((LATENT, LATENT)), _full_spec((1, LATENT)),
                  _full_spec((LATENT, LATENT)), _full_spec((1, LATENT)),
                  _full_spec((1, LATENT)), _full_spec((1, LATENT)),
                  _full_spec((LATENT, LATENT)), _full_spec((LATENT, LATENT))],
        out_specs=[_row_spec(t, LATENT), _row_spec(t, 2 * LATENT)],
        out_shape=[out, outt],
    )(nl, aggs_a, aggs_a, aggs_b, aggs_b, v1n, v1a, c1, v2, c2, v3, c3,
      lns, lnb, wa, wb)


def _decoder_kernel(a_dec, d1, w2, d2, w3, d3):
    def body(a_r, d1_r, w2_r, d2_r, w3_r, d3_r, o_r):
        x = jnp.maximum(a_r[...][:, :LATENT] + d1_r[...], 0.0)
        x = jnp.maximum(_dot(x, w2_r[...]) + d2_r[...], 0.0)
        o_r[...] = _dot(x, w3_r[...]) + d3_r[...]

    t = _NODE_TILE
    k = w3.shape[1]
    return pl.pallas_call(
        body,
        grid=(N_NODES // t,),
        in_specs=[_row_spec(t, 2 * LATENT), _full_spec((1, LATENT)),
                  _full_spec((LATENT, LATENT)), _full_spec((1, LATENT)),
                  _full_spec((LATENT, k)), _full_spec((1, k))],
        out_specs=_row_spec(t, k),
        out_shape=jax.ShapeDtypeStruct((N_NODES, k), jnp.float32),
    )(a_dec, d1, w2, d2, w3, d3)


def _sc_gather(t_tab, s3, r3, n_edges, chunk, nchunk):
    epw = n_edges // _NW
    mesh = plsc.VectorSubcoreMesh(core_axis_name="c", subcore_axis_name="s")
    out = jax.ShapeDtypeStruct((n_edges, 2 * LATENT), jnp.float32)
    ksl = 4
    gpi = 5
    cpi = ksl * gpi
    nit = nchunk // cpi

    @functools.partial(
        pl.kernel, mesh=mesh,
        out_type=[out, out],
        scratch_types=[
            pltpu.VMEM((nchunk, chunk), jnp.int32),
            pltpu.VMEM((nchunk, chunk), jnp.int32),
            pltpu.VMEM((ksl, chunk, 2 * LATENT), jnp.float32),
            pltpu.VMEM((ksl, chunk, 2 * LATENT), jnp.float32),
            pltpu.VMEM((ksl, chunk, 2 * LATENT), jnp.float32),
            pltpu.VMEM((ksl, chunk, 2 * LATENT), jnp.float32),
            pltpu.SemaphoreType.DMA,
            pltpu.SemaphoreType.DMA,
            pltpu.SemaphoreType.DMA,
            pltpu.SemaphoreType.DMA,
        ],
    )
    def k(t_hbm, s_hbm, r_hbm, gs_hbm, gr_hbm,
          sidx, ridx, bufa0, bufa1, bufb0, bufb1, sema, semb, sem_sa, sem_sb):
        wid = lax.axis_index("c") * _NS + lax.axis_index("s")
        pltpu.sync_copy(s_hbm.at[wid], sidx)
        pltpu.sync_copy(r_hbm.at[wid], ridx)
        base = wid * epw
        bufa = (bufa0, bufa1)
        bufb = (bufb0, bufb1)

        def body(it, carry):
            c0 = it * cpi

            def fire_g(g, s):
                ds = []
                for b in range(ksl):
                    j = c0 + g * ksl + b
                    da = pltpu.async_copy(t_hbm.at[sidx.at[j]], bufa[s].at[b], sema)
                    db = pltpu.async_copy(t_hbm.at[ridx.at[j]], bufb[s].at[b], semb)
                    ds.append((da, db))
                return ds

            def fire_s(g, s):
                ds = []
                for b in range(ksl):
                    j = c0 + g * ksl + b
                    off = base + j * chunk
                    da = pltpu.async_copy(bufa[s].at[b],
                                          gs_hbm.at[pl.ds(off, chunk)], sem_sa)
                    db = pltpu.async_copy(bufb[s].at[b],
                                          gr_hbm.at[pl.ds(off, chunk)], sem_sb)
                    ds.append((da, db))
                return ds

            gath = fire_g(0, 0)
            stores_prev = []
            for g in range(gpi):
                s = g % 2
                for da, db in stores_prev:
                    da.wait()
                    db.wait()
                nxt = fire_g(g + 1, 1 - s) if g + 1 < gpi else []
                for da, db in gath:
                    da.wait()
                    db.wait()
                stores_prev = fire_s(g, s)
                gath = nxt
            for da, db in stores_prev:
                da.wait()
                db.wait()
            return carry

        lax.fori_loop(0, nit, body, 0)

        for j in range(nit * cpi, nchunk):
            off = base + j * chunk
            ca = pltpu.async_copy(t_hbm.at[sidx.at[j]], bufa[0].at[0], sema)
            cb = pltpu.async_copy(t_hbm.at[ridx.at[j]], bufb[0].at[0], semb)
            ca.wait()
            cb.wait()
            pltpu.sync_copy(bufa[0].at[0], gs_hbm.at[pl.ds(off, chunk)])
            pltpu.sync_copy(bufb[0].at[0], gr_hbm.at[pl.ds(off, chunk)])

    return k(t_tab, s3, r3)


def _sc_scatter(new_e, r3, zeros, n_edges, chunk, nchunk):
    epw = n_edges // _NW
    mesh = plsc.VectorSubcoreMesh(core_axis_name="c", subcore_axis_name="s")

    ksl = 3
    gpi = 5
    cpi = ksl * gpi
    nit = nchunk // cpi

    @functools.partial(
        pl.kernel, mesh=mesh,
        out_type=jax.ShapeDtypeStruct((_NC, N_NODES, 2 * LATENT), jnp.float32),
        scratch_types=[
            pltpu.VMEM((nchunk, chunk), jnp.int32),
            pltpu.VMEM((ksl, chunk, 2 * LATENT), jnp.float32),
            pltpu.VMEM((ksl, chunk, 2 * LATENT), jnp.float32),
            pltpu.VMEM_SHARED((N_NODES, 2 * LATENT), jnp.float32),
            pltpu.SemaphoreType.DMA,
            pltpu.SemaphoreType.DMA,
        ],
    )
    def k(ne_hbm, r_hbm, z_hbm, out_hbm, ridx, buf0, buf1, agg, sem_l, sem_a):
        cid = lax.axis_index("c")
        sid = lax.axis_index("s")
        wid = cid * _NS + sid

        @pl.when(sid == 0)
        def _():
            pltpu.sync_copy(z_hbm, agg)

        plsc.subcore_barrier()
        pltpu.sync_copy(r_hbm.at[wid], ridx)
        base = wid * epw
        buf = (buf0, buf1)

        def body(it, carry):
            c0 = it * cpi

            def fire_l(g, s):
                ds = []
                for b in range(ksl):
                    j = c0 + g * ksl + b
                    ds.append(pltpu.async_copy(
                        ne_hbm.at[pl.ds(base + j * chunk, chunk)],
                        buf[s].at[b], sem_l))
                return ds

            def fire_a(g, s):
                ds = []
                for b in range(ksl):
                    j = c0 + g * ksl + b
                    ds.append(pltpu.async_copy(
                        buf[s].at[b], agg.at[ridx.at[j]], sem_a, add=True))
                return ds

            loads = fire_l(0, 0)
            adds_prev = []
            for g in range(gpi):
                s = g % 2
                for d in adds_prev:
                    d.wait()
                nxt = fire_l(g + 1, 1 - s) if g + 1 < gpi else []
                for d in loads:
                    d.wait()
                adds_prev = fire_a(g, s)
                loads = nxt
            for d in adds_prev:
                d.wait()
            return carry

        lax.fori_loop(0, nit, body, 0)

        for j in range(nit * cpi, nchunk):
            pltpu.sync_copy(ne_hbm.at[pl.ds(base + j * chunk, chunk)],
                            buf[0].at[0])
            pltpu.sync_copy(buf[0].at[0], agg.at[ridx.at[j]], add=True)
        plsc.subcore_barrier()
        pltpu.sync_copy(agg.at[pl.ds(sid * _NPT, _NPT)],
                        out_hbm.at[cid, pl.ds(sid * _NPT, _NPT)])

        @pl.when(sid == 0)
        def _tail():
            rem = _NS * _NPT
            pltpu.sync_copy(agg.at[pl.ds(rem, N_NODES - rem)],
                            out_hbm.at[cid, pl.ds(rem, N_NODES - rem)])

    return k(new_e, r3, zeros)


def _mlp_params(p):
    (w0, b0), (w1, b1), (w2, b2) = p["layers"]
    lns, lnb = p["ln"]
    return (w0, b0.reshape(1, -1), w1, b1.reshape(1, -1), w2,
            b2.reshape(1, -1), lns.reshape(1, -1), lnb.reshape(1, -1))


def _edge_l1_split(blk):
    w1 = blk["edge"]["layers"][0][0]
    return w1[:LATENT], w1[LATENT:2 * LATENT], w1[2 * LATENT:]


def kernel(node_features, edge_features, params, senders, receivers):
    nf = node_features[0]
    ef = edge_features[0]
    s = senders[0]
    r = receivers[0]
    s3 = [s[h * _EH:(h + 1) * _EH].reshape(_NW, _NCHUNK_H, _CHUNK_H)
          for h in range(_NH)]
    r3 = [r[h * _EH:(h + 1) * _EH].reshape(_NW, _NCHUNK_H, _CHUNK_H)
          for h in range(_NH)]
    zeros = jnp.zeros((N_NODES, 2 * LATENT), jnp.float32)
    blocks = params["blocks"]

    wa, wb, _ = _edge_l1_split(blocks[0])
    en = _mlp_params(params["enc_node"])
    nl, t_tab = _enc_node_kernel(nf, *en, wa, wb)
    ee = _mlp_params(params["enc_edge"])
    el = [None] * _NH

    for i in range(len(blocks)):
        blk = blocks[i]
        _, _, w1c = _edge_l1_split(blk)
        ew = _mlp_params(blk["edge"])
        g = [_sc_gather(t_tab, s3[h], r3[h], _EH, _CHUNK_H, _NCHUNK_H)
             for h in range(_NH)]
        aggs = []
        for h in range(_NH):
            if i == 0:
                ne, el[h] = _edge_enc_mlp_kernel(
                    g[h][0], g[h][1], ef[h * _EH:(h + 1) * _EH], ee, w1c,
                    ew[1], ew[2], ew[3], ew[4], ew[5], ew[6], ew[7])
            else:
                ne, el[h] = _edge_mlp_kernel(g[h][0], g[h][1], el[h], w1c,
                                             ew[1], ew[2], ew[3], ew[4],
                                             ew[5], ew[6], ew[7])
            aggs.append(_sc_scatter(ne, r3[h], zeros, _EH, _CHUNK_H,
                                    _NCHUNK_H))
        if i + 1 < len(blocks):
            wa, wb, _ = _edge_l1_split(blocks[i + 1])
        else:
            wa = params["dec"]["layers"][0][0]
            wb = wa
        nv = blk["node"]["layers"][0][0]
        nw = _mlp_params(blk["node"])
        nl, t_tab = _node_mlp_kernel(
            nl, aggs[0], aggs[1], nv[:LATENT], nv[LATENT:], nw[1], nw[2],
            nw[3], nw[4], nw[5], nw[6], nw[7], wa, wb)

    dec = params["dec"]["layers"]
    out = _decoder_kernel(t_tab, dec[0][1].reshape(1, -1),
                          dec[1][0], dec[1][1].reshape(1, -1),
                          dec[2][0], dec[2][1].reshape(1, -1))
    return out.reshape(1, N_NODES, -1)


# speedup vs baseline: 1.7211x; 1.0037x over previous
import functools

import jax
import jax.numpy as jnp
from jax import lax
from jax.experimental import pallas as pl
from jax.experimental.pallas import tpu as pltpu
from jax.experimental.pallas import tpu_sc as plsc

N_NODES = 10000
N_EDGES = 320000
LATENT = 64

_NC = 2
_NS = 16
_NW = _NC * _NS
_EPW = N_EDGES // _NW
_CHUNK = 80
_NCHUNK = _EPW // _CHUNK
_NPT = 624

_NH = 2
_EH = N_EDGES // _NH
_EPW_H = _EH // _NW
_CHUNK_H = 40
_NCHUNK_H = _EPW_H // _CHUNK_H

_KSLOT = 5
_GPI = 5
_CPI = _KSLOT * _GPI

_EDGE_TILE = 2000
_NODE_TILE = 2000


def _dot(x, w):
    return jnp.dot(x, w, preferred_element_type=jnp.float32)


def _ln(x, scale, bias):
    mu = jnp.mean(x, axis=-1, keepdims=True)
    var = jnp.mean((x - mu) ** 2, axis=-1, keepdims=True)
    return (x - mu) * lax.rsqrt(var + 1e-5) * scale + bias


def _row_spec(tile, width):
    return pl.BlockSpec((tile, width), lambda i: (i, 0))


def _full_spec(shape):
    return pl.BlockSpec(shape, lambda i: (0, 0))


def _enc_node_kernel(nf, w0, b0, w1, b1, w2, b2, lns, lnb, wa, wb):
    def body(nf_r, w0_r, b0_r, w1_r, b1_r, w2_r, b2_r, s_r, t_r, wa_r, wb_r,
             nl_r, t_out_r):
        x = jnp.maximum(_dot(nf_r[...], w0_r[...]) + b0_r[...], 0.0)
        x = jnp.maximum(_dot(x, w1_r[...]) + b1_r[...], 0.0)
        x = _dot(x, w2_r[...]) + b2_r[...]
        nl = _ln(x, s_r[...], t_r[...])
        nl_r[...] = nl
        t_out_r[...] = jnp.concatenate(
            [_dot(nl, wa_r[...]), _dot(nl, wb_r[...])], axis=-1)

    t = _NODE_TILE
    out = jax.ShapeDtypeStruct((N_NODES, LATENT), jnp.float32)
    outt = jax.ShapeDtypeStruct((N_NODES, 2 * LATENT), jnp.float32)
    return pl.pallas_call(
        body,
        grid=(N_NODES // t,),
        in_specs=[_row_spec(t, nf.shape[1]),
                  _full_spec(w0.shape), _full_spec((1, LATENT)),
                  _full_spec(w1.shape), _full_spec((1, LATENT)),
                  _full_spec(w2.shape), _full_spec((1, LATENT)),
                  _full_spec((1, LATENT)), _full_spec((1, LATENT)),
                  _full_spec(wa.shape), _full_spec(wb.shape)],
        out_specs=[_row_spec(t, LATENT), _row_spec(t, 2 * LATENT)],
        out_shape=[out, outt],
    )(nf, w0, b0, w1, b1, w2, b2, lns, lnb, wa, wb)


def _enc_edge_kernel(ef, w0, b0, w1, b1, w2, b2, lns, lnb):
    def body(ef_r, w0_r, b0_r, w1_r, b1_r, w2_r, b2_r, s_r, t_r, el_r):
        x = jnp.maximum(_dot(ef_r[...], w0_r[...]) + b0_r[...], 0.0)
        x = jnp.maximum(_dot(x, w1_r[...]) + b1_r[...], 0.0)
        x = _dot(x, w2_r[...]) + b2_r[...]
        el_r[...] = _ln(x, s_r[...], t_r[...])

    t = _EDGE_TILE
    n = ef.shape[0]
    return pl.pallas_call(
        body,
        grid=(n // t,),
        in_specs=[_row_spec(t, ef.shape[1]),
                  _full_spec(w0.shape), _full_spec((1, LATENT)),
                  _full_spec(w1.shape), _full_spec((1, LATENT)),
                  _full_spec(w2.shape), _full_spec((1, LATENT)),
                  _full_spec((1, LATENT)), _full_spec((1, LATENT))],
        out_specs=_row_spec(t, LATENT),
        out_shape=jax.ShapeDtypeStruct((n, LATENT), jnp.float32),
    )(ef, w0, b0, w1, b1, w2, b2, lns, lnb)


def _edge_mlp_kernel(gs, gr, el, w1c, b1, w2, b2, w3, b3, lns, lnb):
    def body(gs_r, gr_r, el_r, w1c_r, b1_r, w2_r, b2_r, w3_r, b3_r, s_r, t_r,
             ne_r, eo_r):
        elv = el_r[...]
        x = (gs_r[...][:, :LATENT] + gr_r[...][:, LATENT:]
             + _dot(elv, w1c_r[...]) + b1_r[...])
        x = jnp.maximum(x, 0.0)
        x = jnp.maximum(_dot(x, w2_r[...]) + b2_r[...], 0.0)
        x = _dot(x, w3_r[...]) + b3_r[...]
        ne = _ln(x, s_r[...], t_r[...])
        ne_r[...] = jnp.concatenate([ne, jnp.zeros_like(ne)], axis=-1)
        eo_r[...] = elv + ne

    t = _EDGE_TILE
    n = gs.shape[0]
    out = jax.ShapeDtypeStruct((n, LATENT), jnp.float32)
    outp = jax.ShapeDtypeStruct((n, 2 * LATENT), jnp.float32)
    return pl.pallas_call(
        body,
        grid=(n // t,),
        in_specs=[_row_spec(t, 2 * LATENT), _row_spec(t, 2 * LATENT),
                  _row_spec(t, LATENT),
                  _full_spec((LATENT, LATENT)), _full_spec((1, LATENT)),
                  _full_spec((LATENT, LATENT)), _full_spec((1, LATENT)),
                  _full_spec((LATENT, LATENT)), _full_spec((1, LATENT)),
                  _full_spec((1, LATENT)), _full_spec((1, LATENT))],
        out_specs=[_row_spec(t, 2 * LATENT), _row_spec(t, LATENT)],
        out_shape=[outp, out],
    )(gs, gr, el, w1c, b1, w2, b2, w3, b3, lns, lnb)


def _edge_enc_mlp_kernel(gs, gr, ef, enc, w1c, b1, w2, b2, w3, b3, lns, lnb):
    ew0, eb0, ew1, eb1, ew2, eb2, elns, elnb = enc

    def body(gs_r, gr_r, ef_r, ew0_r, eb0_r, ew1_r, eb1_r, ew2_r, eb2_r,
             es_r, et_r, w1c_r, b1_r, w2_r, b2_r, w3_r, b3_r, s_r, t_r,
             ne_r, eo_r):
        e = jnp.maximum(_dot(ef_r[...], ew0_r[...]) + eb0_r[...], 0.0)
        e = jnp.maximum(_dot(e, ew1_r[...]) + eb1_r[...], 0.0)
        e = _dot(e, ew2_r[...]) + eb2_r[...]
        elv = _ln(e, es_r[...], et_r[...])
        x = (gs_r[...][:, :LATENT] + gr_r[...][:, LATENT:]
             + _dot(elv, w1c_r[...]) + b1_r[...])
        x = jnp.maximum(x, 0.0)
        x = jnp.maximum(_dot(x, w2_r[...]) + b2_r[...], 0.0)
        x = _dot(x, w3_r[...]) + b3_r[...]
        ne = _ln(x, s_r[...], t_r[...])
        ne_r[...] = jnp.concatenate([ne, jnp.zeros_like(ne)], axis=-1)
        eo_r[...] = elv + ne

    t = _EDGE_TILE
    n = gs.shape[0]
    out = jax.ShapeDtypeStruct((n, LATENT), jnp.float32)
    outp = jax.ShapeDtypeStruct((n, 2 * LATENT), jnp.float32)
    return pl.pallas_call(
        body,
        grid=(n // t,),
        in_specs=[_row_spec(t, 2 * LATENT), _row_spec(t, 2 * LATENT),
                  _row_spec(t, ef.shape[1]),
                  _full_spec(ew0.shape), _full_spec((1, LATENT)),
                  _full_spec(ew1.shape), _full_spec((1, LATENT)),
                  _full_spec(ew2.shape), _full_spec((1, LATENT)),
                  _full_spec((1, LATENT)), _full_spec((1, LATENT)),
                  _full_spec((LATENT, LATENT)), _full_spec((1, LATENT)),
                  _full_spec((LATENT, LATENT)), _full_spec((1, LATENT)),
                  _full_spec((LATENT, LATENT)), _full_spec((1, LATENT)),
                  _full_spec((1, LATENT)), _full_spec((1, LATENT))],
        out_specs=[_row_spec(t, 2 * LATENT), _row_spec(t, LATENT)],
        out_shape=[outp, out],
    )(gs, gr, ef, ew0, eb0, ew1, eb1, ew2, eb2, elns, elnb,
      w1c, b1, w2, b2, w3, b3, lns, lnb)


def _node_mlp_kernel(nl, aggs_a, aggs_b, v1n, v1a, c1, v2, c2, v3, c3,
                     lns, lnb, wa, wb):
    def body(nl_r, a0_r, a1_r, b0_r, b1_r, v1n_r, v1a_r, c1_r, v2_r, c2_r,
             v3_r, c3_r, s_r, t_r, wa_r, wb_r, no_r, t_out_r):
        nlv = nl_r[...]
        agg = ((a0_r[...][0] + a1_r[...][0])
               + (b0_r[...][0] + b1_r[...][0]))[:, :LATENT]
        x = jnp.maximum(_dot(nlv, v1n_r[...]) + _dot(agg, v1a_r[...]) + c1_r[...], 0.0)
        x = jnp.maximum(_dot(x, v2_r[...]) + c2_r[...], 0.0)
        x = _dot(x, v3_r[...]) + c3_r[...]
        nn = nlv + _ln(x, s_r[...], t_r[...])
        no_r[...] = nn
        t_out_r[...] = jnp.concatenate(
            [_dot(nn, wa_r[...]), _dot(nn, wb_r[...])], axis=-1)

    t = _NODE_TILE
    out = jax.ShapeDtypeStruct((N_NODES, LATENT), jnp.float32)
    outt = jax.ShapeDtypeStruct((N_NODES, 2 * LATENT), jnp.float32)
    return pl.pallas_call(
        body,
        grid=(N_NODES // t,),
        in_specs=[_row_spec(t, LATENT),
                  pl.BlockSpec((1, t, 2 * LATENT), lambda i: (0, i, 0)),
                  pl.BlockSpec((1, t, 2 * LATENT), lambda i: (1, i, 0)),
                  pl.BlockSpec((1, t, 2 * LATENT), lambda i: (0, i, 0)),
                  pl.BlockSpec((1, t, 2 * LATENT), lambda i: (1, i, 0)),
                  _full_spec((LATENT, LATENT)), _full_spec((LATENT, LATENT)),
                  _full_spec((1, LATENT)),
                  _full_spec((LATENT, LATENT)), _full_spec((1, LATENT)),
                  _full_spec((LATENT, LATENT)), _full_spec((1, LATENT)),
                  _full_spec((1, LATENT)), _full_spec((1, LATENT)),
                  _full_spec((LATENT, LATENT)), _full_spec((LATENT, LATENT))],
        out_specs=[_row_spec(t, LATENT), _row_spec(t, 2 * LATENT)],
        out_shape=[out, outt],
    )(nl, aggs_a, aggs_a, aggs_b, aggs_b, v1n, v1a, c1, v2, c2, v3, c3,
      lns, lnb, wa, wb)


def _decoder_kernel(a_dec, d1, w2, d2, w3, d3):
    def body(a_r, d1_r, w2_r, d2_r, w3_r, d3_r, o_r):
        x = jnp.maximum(a_r[...][:, :LATENT] + d1_r[...], 0.0)
        x = jnp.maximum(_dot(x, w2_r[...]) + d2_r[...], 0.0)
        o_r[...] = _dot(x, w3_r[...]) + d3_r[...]

    t = _NODE_TILE
    k = w3.shape[1]
    return pl.pallas_call(
        body,
        grid=(N_NODES // t,),
        in_specs=[_row_spec(t, 2 * LATENT), _full_spec((1, LATENT)),
                  _full_spec((LATENT, LATENT)), _full_spec((1, LATENT)),
                  _full_spec((LATENT, k)), _full_spec((1, k))],
        out_specs=_row_spec(t, k),
        out_shape=jax.ShapeDtypeStruct((N_NODES, k), jnp.float32),
    )(a_dec, d1, w2, d2, w3, d3)


def _sc_gather(t_tab, s1, r1, n_edges, chunk, nchunk):
    epw = n_edges // _NW
    mesh = plsc.VectorSubcoreMesh(core_axis_name="c", subcore_axis_name="s")
    out = jax.ShapeDtypeStruct((n_edges, 2 * LATENT), jnp.float32)
    ksl = 5
    gpi = 5
    cpi = ksl * gpi
    nit = nchunk // cpi

    @functools.partial(
        pl.kernel, mesh=mesh,
        out_type=[out, out],
        scratch_types=[
            pltpu.VMEM((epw,), jnp.int32),
            pltpu.VMEM((epw,), jnp.int32),
            pltpu.VMEM((ksl, chunk, 2 * LATENT), jnp.float32),
            pltpu.VMEM((ksl, chunk, 2 * LATENT), jnp.float32),
            pltpu.VMEM((ksl, chunk, 2 * LATENT), jnp.float32),
            pltpu.VMEM((ksl, chunk, 2 * LATENT), jnp.float32),
            pltpu.SemaphoreType.DMA,
            pltpu.SemaphoreType.DMA,
            pltpu.SemaphoreType.DMA,
            pltpu.SemaphoreType.DMA,
        ],
    )
    def k(t_hbm, s_hbm, r_hbm, gs_hbm, gr_hbm,
          sidx, ridx, bufa0, bufa1, bufb0, bufb1, sema, semb, sem_sa, sem_sb):
        wid = lax.axis_index("c") * _NS + lax.axis_index("s")
        base = wid * epw
        pltpu.sync_copy(s_hbm.at[pl.ds(base, epw)], sidx)
        pltpu.sync_copy(r_hbm.at[pl.ds(base, epw)], ridx)
        bufa = (bufa0, bufa1)
        bufb = (bufb0, bufb1)

        def body(it, carry):
            c0 = it * cpi

            def fire_g(g, s):
                ds = []
                for b in range(ksl):
                    j = c0 + g * ksl + b
                    da = pltpu.async_copy(
                        t_hbm.at[sidx.at[pl.ds(j * chunk, chunk)]],
                        bufa[s].at[b], sema)
                    db = pltpu.async_copy(
                        t_hbm.at[ridx.at[pl.ds(j * chunk, chunk)]],
                        bufb[s].at[b], semb)
                    ds.append((da, db))
                return ds

            def fire_s(g, s):
                ds = []
                for b in range(ksl):
                    j = c0 + g * ksl + b
                    off = base + j * chunk
                    da = pltpu.async_copy(bufa[s].at[b],
                                          gs_hbm.at[pl.ds(off, chunk)], sem_sa)
                    db = pltpu.async_copy(bufb[s].at[b],
                                          gr_hbm.at[pl.ds(off, chunk)], sem_sb)
                    ds.append((da, db))
                return ds

            gath = fire_g(0, 0)
            stores_prev = []
            for g in range(gpi):
                s = g % 2
                for da, db in stores_prev:
                    da.wait()
                    db.wait()
                nxt = fire_g(g + 1, 1 - s) if g + 1 < gpi else []
                for da, db in gath:
                    da.wait()
                    db.wait()
                stores_prev = fire_s(g, s)
                gath = nxt
            for da, db in stores_prev:
                da.wait()
                db.wait()
            return carry

        lax.fori_loop(0, nit, body, 0)

        for j in range(nit * cpi, nchunk):
            off = base + j * chunk
            ca = pltpu.async_copy(t_hbm.at[sidx.at[pl.ds(j * chunk, chunk)]],
                                  bufa[0].at[0], sema)
            cb = pltpu.async_copy(t_hbm.at[ridx.at[pl.ds(j * chunk, chunk)]],
                                  bufb[0].at[0], semb)
            ca.wait()
            cb.wait()
            pltpu.sync_copy(bufa[0].at[0], gs_hbm.at[pl.ds(off, chunk)])
            pltpu.sync_copy(bufb[0].at[0], gr_hbm.at[pl.ds(off, chunk)])

    return k(t_tab, s1, r1)


def _sc_scatter(new_e, r3, zeros, n_edges, chunk, nchunk):
    epw = n_edges // _NW
    mesh = plsc.VectorSubcoreMesh(core_axis_name="c", subcore_axis_name="s")

    ksl = 3
    gpi = 5
    cpi = ksl * gpi
    nit = nchunk // cpi

    @functools.partial(
        pl.kernel, mesh=mesh,
        out_type=jax.ShapeDtypeStruct((_NC, N_NODES, 2 * LATENT), jnp.float32),
        scratch_types=[
            pltpu.VMEM((nchunk, chunk), jnp.int32),
            pltpu.VMEM((ksl, chunk, 2 * LATENT), jnp.float32),
            pltpu.VMEM((ksl, chunk, 2 * LATENT), jnp.float32),
            pltpu.VMEM_SHARED((N_NODES, 2 * LATENT), jnp.float32),
            pltpu.SemaphoreType.DMA,
            pltpu.SemaphoreType.DMA,
        ],
    )
    def k(ne_hbm, r_hbm, z_hbm, out_hbm, ridx, buf0, buf1, agg, sem_l, sem_a):
        cid = lax.axis_index("c")
        sid = lax.axis_index("s")
        wid = cid * _NS + sid

        @pl.when(sid == 0)
        def _():
            pltpu.sync_copy(z_hbm, agg)

        plsc.subcore_barrier()
        pltpu.sync_copy(r_hbm.at[wid], ridx)
        base = wid * epw
        buf = (buf0, buf1)

        def body(it, carry):
            c0 = it * cpi

            def fire_l(g, s):
                ds = []
                for b in range(ksl):
                    j = c0 + g * ksl + b
                    ds.append(pltpu.async_copy(
                        ne_hbm.at[pl.ds(base + j * chunk, chunk)],
                        buf[s].at[b], sem_l))
                return ds

            def fire_a(g, s):
                ds = []
                for b in range(ksl):
                    j = c0 + g * ksl + b
                    ds.append(pltpu.async_copy(
                        buf[s].at[b], agg.at[ridx.at[j]], sem_a, add=True))
                return ds

            loads = fire_l(0, 0)
            adds_prev = []
            for g in range(gpi):
                s = g % 2
                for d in adds_prev:
                    d.wait()
                nxt = fire_l(g + 1, 1 - s) if g + 1 < gpi else []
                for d in loads:
                    d.wait()
                adds_prev = fire_a(g, s)
                loads = nxt
            for d in adds_prev:
                d.wait()
            return carry

        lax.fori_loop(0, nit, body, 0)

        for j in range(nit * cpi, nchunk):
            pltpu.sync_copy(ne_hbm.at[pl.ds(base + j * chunk, chunk)],
                            buf[0].at[0])
            pltpu.sync_copy(buf[0].at[0], agg.at[ridx.at[j]], add=True)
        plsc.subcore_barrier()
        pltpu.sync_copy(agg.at[pl.ds(sid * _NPT, _NPT)],
                        out_hbm.at[cid, pl.ds(sid * _NPT, _NPT)])

        @pl.when(sid == 0)
        def _tail():
            rem = _NS * _NPT
            pltpu.sync_copy(agg.at[pl.ds(rem, N_NODES - rem)],
                            out_hbm.at[cid, pl.ds(rem, N_NODES - rem)])

    return k(new_e, r3, zeros)


def _mlp_params(p):
    (w0, b0), (w1, b1), (w2, b2) = p["layers"]
    lns, lnb = p["ln"]
    return (w0, b0.reshape(1, -1), w1, b1.reshape(1, -1), w2,
            b2.reshape(1, -1), lns.reshape(1, -1), lnb.reshape(1, -1))


def _edge_l1_split(blk):
    w1 = blk["edge"]["layers"][0][0]
    return w1[:LATENT], w1[LATENT:2 * LATENT], w1[2 * LATENT:]


def kernel(node_features, edge_features, params, senders, receivers):
    nf = node_features[0]
    ef = edge_features[0]
    s = senders[0]
    r = receivers[0]
    s1 = [s[h * _EH:(h + 1) * _EH] for h in range(_NH)]
    r1 = [r[h * _EH:(h + 1) * _EH] for h in range(_NH)]
    r3 = [r[h * _EH:(h + 1) * _EH].reshape(_NW, _NCHUNK_H, _CHUNK_H)
          for h in range(_NH)]
    zeros = jnp.zeros((N_NODES, 2 * LATENT), jnp.float32)
    blocks = params["blocks"]

    wa, wb, _ = _edge_l1_split(blocks[0])
    en = _mlp_params(params["enc_node"])
    nl, t_tab = _enc_node_kernel(nf, *en, wa, wb)
    ee = _mlp_params(params["enc_edge"])
    el = [None] * _NH

    for i in range(len(blocks)):
        blk = blocks[i]
        _, _, w1c = _edge_l1_split(blk)
        ew = _mlp_params(blk["edge"])
        g = [_sc_gather(t_tab, s1[h], r1[h], _EH, _CHUNK_H, _NCHUNK_H)
             for h in range(_NH)]
        aggs = []
        for h in range(_NH):
            if i == 0:
                ne, el[h] = _edge_enc_mlp_kernel(
                    g[h][0], g[h][1], ef[h * _EH:(h + 1) * _EH], ee, w1c,
                    ew[1], ew[2], ew[3], ew[4], ew[5], ew[6], ew[7])
            else:
                ne, el[h] = _edge_mlp_kernel(g[h][0], g[h][1], el[h], w1c,
                                             ew[1], ew[2], ew[3], ew[4],
                                             ew[5], ew[6], ew[7])
            aggs.append(_sc_scatter(ne, r3[h], zeros, _EH, _CHUNK_H,
                                    _NCHUNK_H))
        if i + 1 < len(blocks):
            wa, wb, _ = _edge_l1_split(blocks[i + 1])
        else:
            wa = params["dec"]["layers"][0][0]
            wb = wa
        nv = blk["node"]["layers"][0][0]
        nw = _mlp_params(blk["node"])
        nl, t_tab = _node_mlp_kernel(
            nl, aggs[0], aggs[1], nv[:LATENT], nv[LATENT:], nw[1], nw[2],
            nw[3], nw[4], nw[5], nw[6], nw[7], wa, wb)

    dec = params["dec"]["layers"]
    out = _decoder_kernel(t_tab, dec[0][1].reshape(1, -1),
                          dec[1][0], dec[1][1].reshape(1, -1),
                          dec[2][0], dec[2][1].reshape(1, -1))
    return out.reshape(1, N_NODES, -1)


# speedup vs baseline: 1.7704x; 1.0286x over previous
import functools

import jax
import jax.numpy as jnp
from jax import lax
from jax.experimental import pallas as pl
from jax.experimental.pallas import tpu as pltpu
from jax.experimental.pallas import tpu_sc as plsc

N_NODES = 10000
N_EDGES = 320000
LATENT = 64

_NC = 2
_NS = 16
_NW = _NC * _NS
_EPW = N_EDGES // _NW
_CHUNK = 80
_NCHUNK = _EPW // _CHUNK
_NPT = 624

_NH = 2
_EH = N_EDGES // _NH
_EPW_H = _EH // _NW
_CHUNK_H = 40
_NCHUNK_H = _EPW_H // _CHUNK_H

_KSLOT = 5
_GPI = 5
_CPI = _KSLOT * _GPI

_EDGE_TILE = 4000
_NODE_TILE = 2000


def _dot(x, w):
    return jnp.dot(x, w, preferred_element_type=jnp.float32)


def _ln(x, scale, bias):
    mu = jnp.mean(x, axis=-1, keepdims=True)
    var = jnp.mean((x - mu) ** 2, axis=-1, keepdims=True)
    return (x - mu) * lax.rsqrt(var + 1e-5) * scale + bias


def _row_spec(tile, width):
    return pl.BlockSpec((tile, width), lambda i: (i, 0))


def _full_spec(shape):
    return pl.BlockSpec(shape, lambda i: (0, 0))


def _enc_node_kernel(nf, w0, b0, w1, b1, w2, b2, lns, lnb, wa, wb):
    def body(nf_r, w0_r, b0_r, w1_r, b1_r, w2_r, b2_r, s_r, t_r, wa_r, wb_r,
             nl_r, t_out_r):
        x = jnp.maximum(_dot(nf_r[...], w0_r[...]) + b0_r[...], 0.0)
        x = jnp.maximum(_dot(x, w1_r[...]) + b1_r[...], 0.0)
        x = _dot(x, w2_r[...]) + b2_r[...]
        nl = _ln(x, s_r[...], t_r[...])
        nl_r[...] = nl
        t_out_r[...] = jnp.concatenate(
            [_dot(nl, wa_r[...]), _dot(nl, wb_r[...])], axis=-1)

    t = _NODE_TILE
    out = jax.ShapeDtypeStruct((N_NODES, LATENT), jnp.float32)
    outt = jax.ShapeDtypeStruct((N_NODES, 2 * LATENT), jnp.float32)
    return pl.pallas_call(
        body,
        grid=(N_NODES // t,),
        in_specs=[_row_spec(t, nf.shape[1]),
                  _full_spec(w0.shape), _full_spec((1, LATENT)),
                  _full_spec(w1.shape), _full_spec((1, LATENT)),
                  _full_spec(w2.shape), _full_spec((1, LATENT)),
                  _full_spec((1, LATENT)), _full_spec((1, LATENT)),
                  _full_spec(wa.shape), _full_spec(wb.shape)],
        out_specs=[_row_spec(t, LATENT), _row_spec(t, 2 * LATENT)],
        out_shape=[out, outt],
    )(nf, w0, b0, w1, b1, w2, b2, lns, lnb, wa, wb)


def _enc_edge_kernel(ef, w0, b0, w1, b1, w2, b2, lns, lnb):
    def body(ef_r, w0_r, b0_r, w1_r, b1_r, w2_r, b2_r, s_r, t_r, el_r):
        x = jnp.maximum(_dot(ef_r[...], w0_r[...]) + b0_r[...], 0.0)
        x = jnp.maximum(_dot(x, w1_r[...]) + b1_r[...], 0.0)
        x = _dot(x, w2_r[...]) + b2_r[...]
        el_r[...] = _ln(x, s_r[...], t_r[...])

    t = _EDGE_TILE
    n = ef.shape[0]
    return pl.pallas_call(
        body,
        grid=(n // t,),
        in_specs=[_row_spec(t, ef.shape[1]),
                  _full_spec(w0.shape), _full_spec((1, LATENT)),
                  _full_spec(w1.shape), _full_spec((1, LATENT)),
                  _full_spec(w2.shape), _full_spec((1, LATENT)),
                  _full_spec((1, LATENT)), _full_spec((1, LATENT))],
        out_specs=_row_spec(t, LATENT),
        out_shape=jax.ShapeDtypeStruct((n, LATENT), jnp.float32),
    )(ef, w0, b0, w1, b1, w2, b2, lns, lnb)


def _edge_mlp_kernel(gs, gr, el, w1c, b1, w2, b2, w3, b3, lns, lnb):
    def body(gs_r, gr_r, el_r, w1c_r, b1_r, w2_r, b2_r, w3_r, b3_r, s_r, t_r,
             ne_r, eo_r):
        elv = el_r[...]
        x = (gs_r[...][:, :LATENT] + gr_r[...][:, LATENT:]
             + _dot(elv, w1c_r[...]) + b1_r[...])
        x = jnp.maximum(x, 0.0)
        x = jnp.maximum(_dot(x, w2_r[...]) + b2_r[...], 0.0)
        x = _dot(x, w3_r[...]) + b3_r[...]
        ne = _ln(x, s_r[...], t_r[...])
        ne_r[...] = jnp.concatenate([ne, jnp.zeros_like(ne)], axis=-1)
        eo_r[...] = elv + ne

    t = _EDGE_TILE
    n = gs.shape[0]
    out = jax.ShapeDtypeStruct((n, LATENT), jnp.float32)
    outp = jax.ShapeDtypeStruct((n, 2 * LATENT), jnp.float32)
    return pl.pallas_call(
        body,
        grid=(n // t,),
        in_specs=[_row_spec(t, 2 * LATENT), _row_spec(t, 2 * LATENT),
                  _row_spec(t, LATENT),
                  _full_spec((LATENT, LATENT)), _full_spec((1, LATENT)),
                  _full_spec((LATENT, LATENT)), _full_spec((1, LATENT)),
                  _full_spec((LATENT, LATENT)), _full_spec((1, LATENT)),
                  _full_spec((1, LATENT)), _full_spec((1, LATENT))],
        out_specs=[_row_spec(t, 2 * LATENT), _row_spec(t, LATENT)],
        out_shape=[outp, out],
    )(gs, gr, el, w1c, b1, w2, b2, w3, b3, lns, lnb)


def _edge_enc_mlp_kernel(gs, gr, ef, enc, w1c, b1, w2, b2, w3, b3, lns, lnb):
    ew0, eb0, ew1, eb1, ew2, eb2, elns, elnb = enc

    def body(gs_r, gr_r, ef_r, ew0_r, eb0_r, ew1_r, eb1_r, ew2_r, eb2_r,
             es_r, et_r, w1c_r, b1_r, w2_r, b2_r, w3_r, b3_r, s_r, t_r,
             ne_r, eo_r):
        e = jnp.maximum(_dot(ef_r[...], ew0_r[...]) + eb0_r[...], 0.0)
        e = jnp.maximum(_dot(e, ew1_r[...]) + eb1_r[...], 0.0)
        e = _dot(e, ew2_r[...]) + eb2_r[...]
        elv = _ln(e, es_r[...], et_r[...])
        x = (gs_r[...][:, :LATENT] + gr_r[...][:, LATENT:]
             + _dot(elv, w1c_r[...]) + b1_r[...])
        x = jnp.maximum(x, 0.0)
        x = jnp.maximum(_dot(x, w2_r[...]) + b2_r[...], 0.0)
        x = _dot(x, w3_r[...]) + b3_r[...]
        ne = _ln(x, s_r[...], t_r[...])
        ne_r[...] = jnp.concatenate([ne, jnp.zeros_like(ne)], axis=-1)
        eo_r[...] = elv + ne

    t = _EDGE_TILE
    n = gs.shape[0]
    out = jax.ShapeDtypeStruct((n, LATENT), jnp.float32)
    outp = jax.ShapeDtypeStruct((n, 2 * LATENT), jnp.float32)
    return pl.pallas_call(
        body,
        grid=(n // t,),
        in_specs=[_row_spec(t, 2 * LATENT), _row_spec(t, 2 * LATENT),
                  _row_spec(t, ef.shape[1]),
                  _full_spec(ew0.shape), _full_spec((1, LATENT)),
                  _full_spec(ew1.shape), _full_spec((1, LATENT)),
                  _full_spec(ew2.shape), _full_spec((1, LATENT)),
                  _full_spec((1, LATENT)), _full_spec((1, LATENT)),
                  _full_spec((LATENT, LATENT)), _full_spec((1, LATENT)),
                  _full_spec((LATENT, LATENT)), _full_spec((1, LATENT)),
                  _full_spec((LATENT, LATENT)), _full_spec((1, LATENT)),
                  _full_spec((1, LATENT)), _full_spec((1, LATENT))],
        out_specs=[_row_spec(t, 2 * LATENT), _row_spec(t, LATENT)],
        out_shape=[outp, out],
    )(gs, gr, ef, ew0, eb0, ew1, eb1, ew2, eb2, elns, elnb,
      w1c, b1, w2, b2, w3, b3, lns, lnb)


def _node_mlp_kernel(nl, aggs_a, aggs_b, v1n, v1a, c1, v2, c2, v3, c3,
                     lns, lnb, wa, wb):
    def body(nl_r, a0_r, a1_r, b0_r, b1_r, v1n_r, v1a_r, c1_r, v2_r, c2_r,
             v3_r, c3_r, s_r, t_r, wa_r, wb_r, no_r, t_out_r):
        nlv = nl_r[...]
        agg = ((a0_r[...][0] + a1_r[...][0])
               + (b0_r[...][0] + b1_r[...][0]))[:, :LATENT]
        x = jnp.maximum(_dot(nlv, v1n_r[...]) + _dot(agg, v1a_r[...]) + c1_r[...], 0.0)
        x = jnp.maximum(_dot(x, v2_r[...]) + c2_r[...], 0.0)
        x = _dot(x, v3_r[...]) + c3_r[...]
        nn = nlv + _ln(x, s_r[...], t_r[...])
        no_r[...] = nn
        t_out_r[...] = jnp.concatenate(
            [_dot(nn, wa_r[...]), _dot(nn, wb_r[...])], axis=-1)

    t = _NODE_TILE
    out = jax.ShapeDtypeStruct((N_NODES, LATENT), jnp.float32)
    outt = jax.ShapeDtypeStruct((N_NODES, 2 * LATENT), jnp.float32)
    return pl.pallas_call(
        body,
        grid=(N_NODES // t,),
        in_specs=[_row_spec(t, LATENT),
                  pl.BlockSpec((1, t, 2 * LATENT), lambda i: (0, i, 0)),
                  pl.BlockSpec((1, t, 2 * LATENT), lambda i: (1, i, 0)),
                  pl.BlockSpec((1, t, 2 * LATENT), lambda i: (0, i, 0)),
                  pl.BlockSpec((1, t, 2 * LATENT), lambda i: (1, i, 0)),
                  _full_spec((LATENT, LATENT)), _full_spec((LATENT, LATENT)),
                  _full_spec((1, LATENT)),
                  _full_spec((LATENT, LATENT)), _full_spec((1, LATENT)),
                  _full_spec((LATENT, LATENT)), _full_spec((1, LATENT)),
                  _full_spec((1, LATENT)), _full_spec((1, LATENT)),
                  _full_spec((LATENT, LATENT)), _full_spec((LATENT, LATENT))],
        out_specs=[_row_spec(t, LATENT), _row_spec(t, 2 * LATENT)],
        out_shape=[out, outt],
    )(nl, aggs_a, aggs_a, aggs_b, aggs_b, v1n, v1a, c1, v2, c2, v3, c3,
      lns, lnb, wa, wb)


def _decoder_kernel(a_dec, d1, w2, d2, w3, d3):
    def body(a_r, d1_r, w2_r, d2_r, w3_r, d3_r, o_r):
        x = jnp.maximum(a_r[...][:, :LATENT] + d1_r[...], 0.0)
        x = jnp.maximum(_dot(x, w2_r[...]) + d2_r[...], 0.0)
        o_r[...] = _dot(x, w3_r[...]) + d3_r[...]

    t = _NODE_TILE
    k = w3.shape[1]
    return pl.pallas_call(
        body,
        grid=(N_NODES // t,),
        in_specs=[_row_spec(t, 2 * LATENT), _full_spec((1, LATENT)),
                  _full_spec((LATENT, LATENT)), _full_spec((1, LATENT)),
                  _full_spec((LATENT, k)), _full_spec((1, k))],
        out_specs=_row_spec(t, k),
        out_shape=jax.ShapeDtypeStruct((N_NODES, k), jnp.float32),
    )(a_dec, d1, w2, d2, w3, d3)


def _sc_gather(t_tab, s1, r1, n_edges, chunk, nchunk):
    epw = n_edges // _NW
    mesh = plsc.VectorSubcoreMesh(core_axis_name="c", subcore_axis_name="s")
    out = jax.ShapeDtypeStruct((n_edges, 2 * LATENT), jnp.float32)
    ksl = 5
    gpi = 5
    cpi = ksl * gpi
    nit = nchunk // cpi

    @functools.partial(
        pl.kernel, mesh=mesh,
        out_type=[out, out],
        scratch_types=[
            pltpu.VMEM((epw,), jnp.int32),
            pltpu.VMEM((epw,), jnp.int32),
            pltpu.VMEM((ksl, chunk, 2 * LATENT), jnp.float32),
            pltpu.VMEM((ksl, chunk, 2 * LATENT), jnp.float32),
            pltpu.VMEM((ksl, chunk, 2 * LATENT), jnp.float32),
            pltpu.VMEM((ksl, chunk, 2 * LATENT), jnp.float32),
            pltpu.SemaphoreType.DMA,
            pltpu.SemaphoreType.DMA,
            pltpu.SemaphoreType.DMA,
            pltpu.SemaphoreType.DMA,
        ],
    )
    def k(t_hbm, s_hbm, r_hbm, gs_hbm, gr_hbm,
          sidx, ridx, bufa0, bufa1, bufb0, bufb1, sema, semb, sem_sa, sem_sb):
        wid = lax.axis_index("c") * _NS + lax.axis_index("s")
        base = wid * epw
        pltpu.sync_copy(s_hbm.at[pl.ds(base, epw)], sidx)
        pltpu.sync_copy(r_hbm.at[pl.ds(base, epw)], ridx)
        bufa = (bufa0, bufa1)
        bufb = (bufb0, bufb1)

        def body(it, carry):
            c0 = it * cpi

            def fire_g(g, s):
                ds = []
                for b in range(ksl):
                    j = c0 + g * ksl + b
                    da = pltpu.async_copy(
                        t_hbm.at[sidx.at[pl.ds(j * chunk, chunk)]],
                        bufa[s].at[b], sema)
                    db = pltpu.async_copy(
                        t_hbm.at[ridx.at[pl.ds(j * chunk, chunk)]],
                        bufb[s].at[b], semb)
                    ds.append((da, db))
                return ds

            def fire_s(g, s):
                ds = []
                for b in range(ksl):
                    j = c0 + g * ksl + b
                    off = base + j * chunk
                    da = pltpu.async_copy(bufa[s].at[b],
                                          gs_hbm.at[pl.ds(off, chunk)], sem_sa)
                    db = pltpu.async_copy(bufb[s].at[b],
                                          gr_hbm.at[pl.ds(off, chunk)], sem_sb)
                    ds.append((da, db))
                return ds

            gath = fire_g(0, 0)
            stores_prev = []
            for g in range(gpi):
                s = g % 2
                for da, db in stores_prev:
                    da.wait()
                    db.wait()
                nxt = fire_g(g + 1, 1 - s) if g + 1 < gpi else []
                for da, db in gath:
                    da.wait()
                    db.wait()
                stores_prev = fire_s(g, s)
                gath = nxt
            for da, db in stores_prev:
                da.wait()
                db.wait()
            return carry

        lax.fori_loop(0, nit, body, 0)

        for j in range(nit * cpi, nchunk):
            off = base + j * chunk
            ca = pltpu.async_copy(t_hbm.at[sidx.at[pl.ds(j * chunk, chunk)]],
                                  bufa[0].at[0], sema)
            cb = pltpu.async_copy(t_hbm.at[ridx.at[pl.ds(j * chunk, chunk)]],
                                  bufb[0].at[0], semb)
            ca.wait()
            cb.wait()
            pltpu.sync_copy(bufa[0].at[0], gs_hbm.at[pl.ds(off, chunk)])
            pltpu.sync_copy(bufb[0].at[0], gr_hbm.at[pl.ds(off, chunk)])

    return k(t_tab, s1, r1)


def _sc_scatter(new_e, r3, zeros, n_edges, chunk, nchunk):
    epw = n_edges // _NW
    mesh = plsc.VectorSubcoreMesh(core_axis_name="c", subcore_axis_name="s")

    ksl = 3
    gpi = 5
    cpi = ksl * gpi
    nit = nchunk // cpi

    @functools.partial(
        pl.kernel, mesh=mesh,
        out_type=jax.ShapeDtypeStruct((_NC, N_NODES, 2 * LATENT), jnp.float32),
        scratch_types=[
            pltpu.VMEM((nchunk, chunk), jnp.int32),
            pltpu.VMEM((ksl, chunk, 2 * LATENT), jnp.float32),
            pltpu.VMEM((ksl, chunk, 2 * LATENT), jnp.float32),
            pltpu.VMEM_SHARED((N_NODES, 2 * LATENT), jnp.float32),
            pltpu.SemaphoreType.DMA,
            pltpu.SemaphoreType.DMA,
        ],
    )
    def k(ne_hbm, r_hbm, z_hbm, out_hbm, ridx, buf0, buf1, agg, sem_l, sem_a):
        cid = lax.axis_index("c")
        sid = lax.axis_index("s")
        wid = cid * _NS + sid

        @pl.when(sid == 0)
        def _():
            pltpu.sync_copy(z_hbm, agg)

        plsc.subcore_barrier()
        pltpu.sync_copy(r_hbm.at[wid], ridx)
        base = wid * epw
        buf = (buf0, buf1)

        def body(it, carry):
            c0 = it * cpi

            def fire_l(g, s):
                ds = []
                for b in range(ksl):
                    j = c0 + g * ksl + b
                    ds.append(pltpu.async_copy(
                        ne_hbm.at[pl.ds(base + j * chunk, chunk)],
                        buf[s].at[b], sem_l))
                return ds

            def fire_a(g, s):
                ds = []
                for b in range(ksl):
                    j = c0 + g * ksl + b
                    ds.append(pltpu.async_copy(
                        buf[s].at[b], agg.at[ridx.at[j]], sem_a, add=True))
                return ds

            loads = fire_l(0, 0)
            adds_prev = []
            for g in range(gpi):
                s = g % 2
                for d in adds_prev:
                    d.wait()
                nxt = fire_l(g + 1, 1 - s) if g + 1 < gpi else []
                for d in loads:
                    d.wait()
                adds_prev = fire_a(g, s)
                loads = nxt
            for d in adds_prev:
                d.wait()
            return carry

        lax.fori_loop(0, nit, body, 0)

        for j in range(nit * cpi, nchunk):
            pltpu.sync_copy(ne_hbm.at[pl.ds(base + j * chunk, chunk)],
                            buf[0].at[0])
            pltpu.sync_copy(buf[0].at[0], agg.at[ridx.at[j]], add=True)
        plsc.subcore_barrier()
        pltpu.sync_copy(agg.at[pl.ds(sid * _NPT, _NPT)],
                        out_hbm.at[cid, pl.ds(sid * _NPT, _NPT)])

        @pl.when(sid == 0)
        def _tail():
            rem = _NS * _NPT
            pltpu.sync_copy(agg.at[pl.ds(rem, N_NODES - rem)],
                            out_hbm.at[cid, pl.ds(rem, N_NODES - rem)])

    return k(new_e, r3, zeros)


def _mlp_params(p):
    (w0, b0), (w1, b1), (w2, b2) = p["layers"]
    lns, lnb = p["ln"]
    return (w0, b0.reshape(1, -1), w1, b1.reshape(1, -1), w2,
            b2.reshape(1, -1), lns.reshape(1, -1), lnb.reshape(1, -1))


def _edge_l1_split(blk):
    w1 = blk["edge"]["layers"][0][0]
    return w1[:LATENT], w1[LATENT:2 * LATENT], w1[2 * LATENT:]


def kernel(node_features, edge_features, params, senders, receivers):
    nf = node_features[0]
    ef = edge_features[0]
    s = senders[0]
    r = receivers[0]
    s1 = [s[h * _EH:(h + 1) * _EH] for h in range(_NH)]
    r1 = [r[h * _EH:(h + 1) * _EH] for h in range(_NH)]
    r3 = [r[h * _EH:(h + 1) * _EH].reshape(_NW, _NCHUNK_H, _CHUNK_H)
          for h in range(_NH)]
    zeros = jnp.zeros((N_NODES, 2 * LATENT), jnp.float32)
    blocks = params["blocks"]

    wa, wb, _ = _edge_l1_split(blocks[0])
    en = _mlp_params(params["enc_node"])
    nl, t_tab = _enc_node_kernel(nf, *en, wa, wb)
    ee = _mlp_params(params["enc_edge"])
    el = [None] * _NH

    for i in range(len(blocks)):
        blk = blocks[i]
        _, _, w1c = _edge_l1_split(blk)
        ew = _mlp_params(blk["edge"])
        g = [_sc_gather(t_tab, s1[h], r1[h], _EH, _CHUNK_H, _NCHUNK_H)
             for h in range(_NH)]
        aggs = []
        for h in range(_NH):
            if i == 0:
                ne, el[h] = _edge_enc_mlp_kernel(
                    g[h][0], g[h][1], ef[h * _EH:(h + 1) * _EH], ee, w1c,
                    ew[1], ew[2], ew[3], ew[4], ew[5], ew[6], ew[7])
            else:
                ne, el[h] = _edge_mlp_kernel(g[h][0], g[h][1], el[h], w1c,
                                             ew[1], ew[2], ew[3], ew[4],
                                             ew[5], ew[6], ew[7])
            aggs.append(_sc_scatter(ne, r3[h], zeros, _EH, _CHUNK_H,
                                    _NCHUNK_H))
        if i + 1 < len(blocks):
            wa, wb, _ = _edge_l1_split(blocks[i + 1])
        else:
            wa = params["dec"]["layers"][0][0]
            wb = wa
        nv = blk["node"]["layers"][0][0]
        nw = _mlp_params(blk["node"])
        nl, t_tab = _node_mlp_kernel(
            nl, aggs[0], aggs[1], nv[:LATENT], nv[LATENT:], nw[1], nw[2],
            nw[3], nw[4], nw[5], nw[6], nw[7], wa, wb)

    dec = params["dec"]["layers"]
    out = _decoder_kernel(t_tab, dec[0][1].reshape(1, -1),
                          dec[1][0], dec[1][1].reshape(1, -1),
                          dec[2][0], dec[2][1].reshape(1, -1))
    return out.reshape(1, N_NODES, -1)


# speedup vs baseline: 1.7736x; 1.0018x over previous
import functools

import jax
import jax.numpy as jnp
from jax import lax
from jax.experimental import pallas as pl
from jax.experimental.pallas import tpu as pltpu
from jax.experimental.pallas import tpu_sc as plsc

N_NODES = 10000
N_EDGES = 320000
LATENT = 64

_NC = 2
_NS = 16
_NW = _NC * _NS
_EPW = N_EDGES // _NW
_CHUNK = 80
_NCHUNK = _EPW // _CHUNK
_NPT = 624

_NH = 2
_EH = N_EDGES // _NH
_EPW_H = _EH // _NW
_CHUNK_H = 40
_NCHUNK_H = _EPW_H // _CHUNK_H

_KSLOT = 5
_GPI = 5
_CPI = _KSLOT * _GPI

_EDGE_TILE = 8000
_NODE_TILE = 2000


def _dot(x, w):
    return jnp.dot(x, w, preferred_element_type=jnp.float32)


def _ln(x, scale, bias):
    mu = jnp.mean(x, axis=-1, keepdims=True)
    var = jnp.mean((x - mu) ** 2, axis=-1, keepdims=True)
    return (x - mu) * lax.rsqrt(var + 1e-5) * scale + bias


def _row_spec(tile, width):
    return pl.BlockSpec((tile, width), lambda i: (i, 0))


def _full_spec(shape):
    return pl.BlockSpec(shape, lambda i: (0, 0))


def _enc_node_kernel(nf, w0, b0, w1, b1, w2, b2, lns, lnb, wa, wb):
    def body(nf_r, w0_r, b0_r, w1_r, b1_r, w2_r, b2_r, s_r, t_r, wa_r, wb_r,
             nl_r, t_out_r):
        x = jnp.maximum(_dot(nf_r[...], w0_r[...]) + b0_r[...], 0.0)
        x = jnp.maximum(_dot(x, w1_r[...]) + b1_r[...], 0.0)
        x = _dot(x, w2_r[...]) + b2_r[...]
        nl = _ln(x, s_r[...], t_r[...])
        nl_r[...] = nl
        t_out_r[...] = jnp.concatenate(
            [_dot(nl, wa_r[...]), _dot(nl, wb_r[...])], axis=-1)

    t = _NODE_TILE
    out = jax.ShapeDtypeStruct((N_NODES, LATENT), jnp.float32)
    outt = jax.ShapeDtypeStruct((N_NODES, 2 * LATENT), jnp.float32)
    return pl.pallas_call(
        body,
        grid=(N_NODES // t,),
        in_specs=[_row_spec(t, nf.shape[1]),
                  _full_spec(w0.shape), _full_spec((1, LATENT)),
                  _full_spec(w1.shape), _full_spec((1, LATENT)),
                  _full_spec(w2.shape), _full_spec((1, LATENT)),
                  _full_spec((1, LATENT)), _full_spec((1, LATENT)),
                  _full_spec(wa.shape), _full_spec(wb.shape)],
        out_specs=[_row_spec(t, LATENT), _row_spec(t, 2 * LATENT)],
        out_shape=[out, outt],
    )(nf, w0, b0, w1, b1, w2, b2, lns, lnb, wa, wb)


def _enc_edge_kernel(ef, w0, b0, w1, b1, w2, b2, lns, lnb):
    def body(ef_r, w0_r, b0_r, w1_r, b1_r, w2_r, b2_r, s_r, t_r, el_r):
        x = jnp.maximum(_dot(ef_r[...], w0_r[...]) + b0_r[...], 0.0)
        x = jnp.maximum(_dot(x, w1_r[...]) + b1_r[...], 0.0)
        x = _dot(x, w2_r[...]) + b2_r[...]
        el_r[...] = _ln(x, s_r[...], t_r[...])

    t = _EDGE_TILE
    n = ef.shape[0]
    return pl.pallas_call(
        body,
        grid=(n // t,),
        in_specs=[_row_spec(t, ef.shape[1]),
                  _full_spec(w0.shape), _full_spec((1, LATENT)),
                  _full_spec(w1.shape), _full_spec((1, LATENT)),
                  _full_spec(w2.shape), _full_spec((1, LATENT)),
                  _full_spec((1, LATENT)), _full_spec((1, LATENT))],
        out_specs=_row_spec(t, LATENT),
        out_shape=jax.ShapeDtypeStruct((n, LATENT), jnp.float32),
    )(ef, w0, b0, w1, b1, w2, b2, lns, lnb)


def _edge_mlp_kernel(gs, gr, el, w1c, b1, w2, b2, w3, b3, lns, lnb):
    def body(gs_r, gr_r, el_r, w1c_r, b1_r, w2_r, b2_r, w3_r, b3_r, s_r, t_r,
             ne_r, eo_r):
        elv = el_r[...]
        x = (gs_r[...][:, :LATENT] + gr_r[...][:, LATENT:]
             + _dot(elv, w1c_r[...]) + b1_r[...])
        x = jnp.maximum(x, 0.0)
        x = jnp.maximum(_dot(x, w2_r[...]) + b2_r[...], 0.0)
        x = _dot(x, w3_r[...]) + b3_r[...]
        ne = _ln(x, s_r[...], t_r[...])
        ne_r[...] = jnp.concatenate([ne, jnp.zeros_like(ne)], axis=-1)
        eo_r[...] = elv + ne

    t = _EDGE_TILE
    n = gs.shape[0]
    out = jax.ShapeDtypeStruct((n, LATENT), jnp.float32)
    outp = jax.ShapeDtypeStruct((n, 2 * LATENT), jnp.float32)
    return pl.pallas_call(
        body,
        grid=(n // t,),
        in_specs=[_row_spec(t, 2 * LATENT), _row_spec(t, 2 * LATENT),
                  _row_spec(t, LATENT),
                  _full_spec((LATENT, LATENT)), _full_spec((1, LATENT)),
                  _full_spec((LATENT, LATENT)), _full_spec((1, LATENT)),
                  _full_spec((LATENT, LATENT)), _full_spec((1, LATENT)),
                  _full_spec((1, LATENT)), _full_spec((1, LATENT))],
        out_specs=[_row_spec(t, 2 * LATENT), _row_spec(t, LATENT)],
        out_shape=[outp, out],
    )(gs, gr, el, w1c, b1, w2, b2, w3, b3, lns, lnb)


def _edge_enc_mlp_kernel(gs, gr, ef, enc, w1c, b1, w2, b2, w3, b3, lns, lnb):
    ew0, eb0, ew1, eb1, ew2, eb2, elns, elnb = enc

    def body(gs_r, gr_r, ef_r, ew0_r, eb0_r, ew1_r, eb1_r, ew2_r, eb2_r,
             es_r, et_r, w1c_r, b1_r, w2_r, b2_r, w3_r, b3_r, s_r, t_r,
             ne_r, eo_r):
        e = jnp.maximum(_dot(ef_r[...], ew0_r[...]) + eb0_r[...], 0.0)
        e = jnp.maximum(_dot(e, ew1_r[...]) + eb1_r[...], 0.0)
        e = _dot(e, ew2_r[...]) + eb2_r[...]
        elv = _ln(e, es_r[...], et_r[...])
        x = (gs_r[...][:, :LATENT] + gr_r[...][:, LATENT:]
             + _dot(elv, w1c_r[...]) + b1_r[...])
        x = jnp.maximum(x, 0.0)
        x = jnp.maximum(_dot(x, w2_r[...]) + b2_r[...], 0.0)
        x = _dot(x, w3_r[...]) + b3_r[...]
        ne = _ln(x, s_r[...], t_r[...])
        ne_r[...] = jnp.concatenate([ne, jnp.zeros_like(ne)], axis=-1)
        eo_r[...] = elv + ne

    t = _EDGE_TILE
    n = gs.shape[0]
    out = jax.ShapeDtypeStruct((n, LATENT), jnp.float32)
    outp = jax.ShapeDtypeStruct((n, 2 * LATENT), jnp.float32)
    return pl.pallas_call(
        body,
        grid=(n // t,),
        in_specs=[_row_spec(t, 2 * LATENT), _row_spec(t, 2 * LATENT),
                  _row_spec(t, ef.shape[1]),
                  _full_spec(ew0.shape), _full_spec((1, LATENT)),
                  _full_spec(ew1.shape), _full_spec((1, LATENT)),
                  _full_spec(ew2.shape), _full_spec((1, LATENT)),
                  _full_spec((1, LATENT)), _full_spec((1, LATENT)),
                  _full_spec((LATENT, LATENT)), _full_spec((1, LATENT)),
                  _full_spec((LATENT, LATENT)), _full_spec((1, LATENT)),
                  _full_spec((LATENT, LATENT)), _full_spec((1, LATENT)),
                  _full_spec((1, LATENT)), _full_spec((1, LATENT))],
        out_specs=[_row_spec(t, 2 * LATENT), _row_spec(t, LATENT)],
        out_shape=[outp, out],
    )(gs, gr, ef, ew0, eb0, ew1, eb1, ew2, eb2, elns, elnb,
      w1c, b1, w2, b2, w3, b3, lns, lnb)


def _node_mlp_kernel(nl, aggs_a, aggs_b, v1n, v1a, c1, v2, c2, v3, c3,
                     lns, lnb, wa, wb):
    def body(nl_r, a0_r, a1_r, b0_r, b1_r, v1n_r, v1a_r, c1_r, v2_r, c2_r,
             v3_r, c3_r, s_r, t_r, wa_r, wb_r, no_r, t_out_r):
        nlv = nl_r[...]
        agg = ((a0_r[...][0] + a1_r[...][0])
               + (b0_r[...][0] + b1_r[...][0]))[:, :LATENT]
        x = jnp.maximum(_dot(nlv, v1n_r[...]) + _dot(agg, v1a_r[...]) + c1_r[...], 0.0)
        x = jnp.maximum(_dot(x, v2_r[...]) + c2_r[...], 0.0)
        x = _dot(x, v3_r[...]) + c3_r[...]
        nn = nlv + _ln(x, s_r[...], t_r[...])
        no_r[...] = nn
        t_out_r[...] = jnp.concatenate(
            [_dot(nn, wa_r[...]), _dot(nn, wb_r[...])], axis=-1)

    t = _NODE_TILE
    out = jax.ShapeDtypeStruct((N_NODES, LATENT), jnp.float32)
    outt = jax.ShapeDtypeStruct((N_NODES, 2 * LATENT), jnp.float32)
    return pl.pallas_call(
        body,
        grid=(N_NODES // t,),
        in_specs=[_row_spec(t, LATENT),
                  pl.BlockSpec((1, t, 2 * LATENT), lambda i: (0, i, 0)),
                  pl.BlockSpec((1, t, 2 * LATENT), lambda i: (1, i, 0)),
                  pl.BlockSpec((1, t, 2 * LATENT), lambda i: (0, i, 0)),
                  pl.BlockSpec((1, t, 2 * LATENT), lambda i: (1, i, 0)),
                  _full_spec((LATENT, LATENT)), _full_spec((LATENT, LATENT)),
                  _full_spec((1, LATENT)),
                  _full_spec((LATENT, LATENT)), _full_spec((1, LATENT)),
                  _full_spec((LATENT, LATENT)), _full_spec((1, LATENT)),
                  _full_spec((1, LATENT)), _full_spec((1, LATENT)),
                  _full_spec((LATENT, LATENT)), _full_spec((LATENT, LATENT))],
        out_specs=[_row_spec(t, LATENT), _row_spec(t, 2 * LATENT)],
        out_shape=[out, outt],
    )(nl, aggs_a, aggs_a, aggs_b, aggs_b, v1n, v1a, c1, v2, c2, v3, c3,
      lns, lnb, wa, wb)


def _decoder_kernel(a_dec, d1, w2, d2, w3, d3):
    def body(a_r, d1_r, w2_r, d2_r, w3_r, d3_r, o_r):
        x = jnp.maximum(a_r[...][:, :LATENT] + d1_r[...], 0.0)
        x = jnp.maximum(_dot(x, w2_r[...]) + d2_r[...], 0.0)
        o_r[...] = _dot(x, w3_r[...]) + d3_r[...]

    t = _NODE_TILE
    k = w3.shape[1]
    return pl.pallas_call(
        body,
        grid=(N_NODES // t,),
        in_specs=[_row_spec(t, 2 * LATENT), _full_spec((1, LATENT)),
                  _full_spec((LATENT, LATENT)), _full_spec((1, LATENT)),
                  _full_spec((LATENT, k)), _full_spec((1, k))],
        out_specs=_row_spec(t, k),
        out_shape=jax.ShapeDtypeStruct((N_NODES, k), jnp.float32),
    )(a_dec, d1, w2, d2, w3, d3)


def _sc_gather(t_tab, s1, r1, n_edges, chunk, nchunk):
    epw = n_edges // _NW
    mesh = plsc.VectorSubcoreMesh(core_axis_name="c", subcore_axis_name="s")
    out = jax.ShapeDtypeStruct((n_edges, 2 * LATENT), jnp.float32)
    ksl = 5
    gpi = 5
    cpi = ksl * gpi
    nit = nchunk // cpi

    @functools.partial(
        pl.kernel, mesh=mesh,
        out_type=[out, out],
        scratch_types=[
            pltpu.VMEM((epw,), jnp.int32),
            pltpu.VMEM((epw,), jnp.int32),
            pltpu.VMEM((ksl, chunk, 2 * LATENT), jnp.float32),
            pltpu.VMEM((ksl, chunk, 2 * LATENT), jnp.float32),
            pltpu.VMEM((ksl, chunk, 2 * LATENT), jnp.float32),
            pltpu.VMEM((ksl, chunk, 2 * LATENT), jnp.float32),
            pltpu.SemaphoreType.DMA,
            pltpu.SemaphoreType.DMA,
            pltpu.SemaphoreType.DMA,
            pltpu.SemaphoreType.DMA,
        ],
    )
    def k(t_hbm, s_hbm, r_hbm, gs_hbm, gr_hbm,
          sidx, ridx, bufa0, bufa1, bufb0, bufb1, sema, semb, sem_sa, sem_sb):
        wid = lax.axis_index("c") * _NS + lax.axis_index("s")
        base = wid * epw
        pltpu.sync_copy(s_hbm.at[pl.ds(base, epw)], sidx)
        pltpu.sync_copy(r_hbm.at[pl.ds(base, epw)], ridx)
        bufa = (bufa0, bufa1)
        bufb = (bufb0, bufb1)

        def body(it, carry):
            c0 = it * cpi

            def fire_g(g, s):
                ds = []
                for b in range(ksl):
                    j = c0 + g * ksl + b
                    da = pltpu.async_copy(
                        t_hbm.at[sidx.at[pl.ds(j * chunk, chunk)]],
                        bufa[s].at[b], sema)
                    db = pltpu.async_copy(
                        t_hbm.at[ridx.at[pl.ds(j * chunk, chunk)]],
                        bufb[s].at[b], semb)
                    ds.append((da, db))
                return ds

            def fire_s(g, s):
                ds = []
                for b in range(ksl):
                    j = c0 + g * ksl + b
                    off = base + j * chunk
                    da = pltpu.async_copy(bufa[s].at[b],
                                          gs_hbm.at[pl.ds(off, chunk)], sem_sa)
                    db = pltpu.async_copy(bufb[s].at[b],
                                          gr_hbm.at[pl.ds(off, chunk)], sem_sb)
                    ds.append((da, db))
                return ds

            gath = fire_g(0, 0)
            stores_prev = []
            for g in range(gpi):
                s = g % 2
                for da, db in stores_prev:
                    da.wait()
                    db.wait()
                nxt = fire_g(g + 1, 1 - s) if g + 1 < gpi else []
                for da, db in gath:
                    da.wait()
                    db.wait()
                stores_prev = fire_s(g, s)
                gath = nxt
            for da, db in stores_prev:
                da.wait()
                db.wait()
            return carry

        lax.fori_loop(0, nit, body, 0)

        for j in range(nit * cpi, nchunk):
            off = base + j * chunk
            ca = pltpu.async_copy(t_hbm.at[sidx.at[pl.ds(j * chunk, chunk)]],
                                  bufa[0].at[0], sema)
            cb = pltpu.async_copy(t_hbm.at[ridx.at[pl.ds(j * chunk, chunk)]],
                                  bufb[0].at[0], semb)
            ca.wait()
            cb.wait()
            pltpu.sync_copy(bufa[0].at[0], gs_hbm.at[pl.ds(off, chunk)])
            pltpu.sync_copy(bufb[0].at[0], gr_hbm.at[pl.ds(off, chunk)])

    return k(t_tab, s1, r1)


def _sc_scatter(new_e, r3, zeros, n_edges, chunk, nchunk):
    epw = n_edges // _NW
    mesh = plsc.VectorSubcoreMesh(core_axis_name="c", subcore_axis_name="s")

    ksl = 3
    gpi = 5
    cpi = ksl * gpi
    nit = nchunk // cpi

    @functools.partial(
        pl.kernel, mesh=mesh,
        out_type=jax.ShapeDtypeStruct((_NC, N_NODES, 2 * LATENT), jnp.float32),
        scratch_types=[
            pltpu.VMEM((nchunk, chunk), jnp.int32),
            pltpu.VMEM((ksl, chunk, 2 * LATENT), jnp.float32),
            pltpu.VMEM((ksl, chunk, 2 * LATENT), jnp.float32),
            pltpu.VMEM_SHARED((N_NODES, 2 * LATENT), jnp.float32),
            pltpu.SemaphoreType.DMA,
            pltpu.SemaphoreType.DMA,
        ],
    )
    def k(ne_hbm, r_hbm, z_hbm, out_hbm, ridx, buf0, buf1, agg, sem_l, sem_a):
        cid = lax.axis_index("c")
        sid = lax.axis_index("s")
        wid = cid * _NS + sid

        @pl.when(sid == 0)
        def _():
            pltpu.sync_copy(z_hbm, agg)

        plsc.subcore_barrier()
        pltpu.sync_copy(r_hbm.at[wid], ridx)
        base = wid * epw
        buf = (buf0, buf1)

        def body(it, carry):
            c0 = it * cpi

            def fire_l(g, s):
                ds = []
                for b in range(ksl):
                    j = c0 + g * ksl + b
                    ds.append(pltpu.async_copy(
                        ne_hbm.at[pl.ds(base + j * chunk, chunk)],
                        buf[s].at[b], sem_l))
                return ds

            def fire_a(g, s):
                ds = []
                for b in range(ksl):
                    j = c0 + g * ksl + b
                    ds.append(pltpu.async_copy(
                        buf[s].at[b], agg.at[ridx.at[j]], sem_a, add=True))
                return ds

            loads = fire_l(0, 0)
            adds_prev = []
            for g in range(gpi):
                s = g % 2
                for d in adds_prev:
                    d.wait()
                nxt = fire_l(g + 1, 1 - s) if g + 1 < gpi else []
                for d in loads:
                    d.wait()
                adds_prev = fire_a(g, s)
                loads = nxt
            for d in adds_prev:
                d.wait()
            return carry

        lax.fori_loop(0, nit, body, 0)

        for j in range(nit * cpi, nchunk):
            pltpu.sync_copy(ne_hbm.at[pl.ds(base + j * chunk, chunk)],
                            buf[0].at[0])
            pltpu.sync_copy(buf[0].at[0], agg.at[ridx.at[j]], add=True)
        plsc.subcore_barrier()
        pltpu.sync_copy(agg.at[pl.ds(sid * _NPT, _NPT)],
                        out_hbm.at[cid, pl.ds(sid * _NPT, _NPT)])

        @pl.when(sid == 0)
        def _tail():
            rem = _NS * _NPT
            pltpu.sync_copy(agg.at[pl.ds(rem, N_NODES - rem)],
                            out_hbm.at[cid, pl.ds(rem, N_NODES - rem)])

    return k(new_e, r3, zeros)


def _mlp_params(p):
    (w0, b0), (w1, b1), (w2, b2) = p["layers"]
    lns, lnb = p["ln"]
    return (w0, b0.reshape(1, -1), w1, b1.reshape(1, -1), w2,
            b2.reshape(1, -1), lns.reshape(1, -1), lnb.reshape(1, -1))


def _edge_l1_split(blk):
    w1 = blk["edge"]["layers"][0][0]
    return w1[:LATENT], w1[LATENT:2 * LATENT], w1[2 * LATENT:]


def kernel(node_features, edge_features, params, senders, receivers):
    nf = node_features[0]
    ef = edge_features[0]
    s = senders[0]
    r = receivers[0]
    s1 = [s[h * _EH:(h + 1) * _EH] for h in range(_NH)]
    r1 = [r[h * _EH:(h + 1) * _EH] for h in range(_NH)]
    r3 = [r[h * _EH:(h + 1) * _EH].reshape(_NW, _NCHUNK_H, _CHUNK_H)
          for h in range(_NH)]
    zeros = jnp.zeros((N_NODES, 2 * LATENT), jnp.float32)
    blocks = params["blocks"]

    wa, wb, _ = _edge_l1_split(blocks[0])
    en = _mlp_params(params["enc_node"])
    nl, t_tab = _enc_node_kernel(nf, *en, wa, wb)
    ee = _mlp_params(params["enc_edge"])
    el = [None] * _NH

    for i in range(len(blocks)):
        blk = blocks[i]
        _, _, w1c = _edge_l1_split(blk)
        ew = _mlp_params(blk["edge"])
        g = [_sc_gather(t_tab, s1[h], r1[h], _EH, _CHUNK_H, _NCHUNK_H)
             for h in range(_NH)]
        aggs = []
        for h in range(_NH):
            if i == 0:
                ne, el[h] = _edge_enc_mlp_kernel(
                    g[h][0], g[h][1], ef[h * _EH:(h + 1) * _EH], ee, w1c,
                    ew[1], ew[2], ew[3], ew[4], ew[5], ew[6], ew[7])
            else:
                ne, el[h] = _edge_mlp_kernel(g[h][0], g[h][1], el[h], w1c,
                                             ew[1], ew[2], ew[3], ew[4],
                                             ew[5], ew[6], ew[7])
            aggs.append(_sc_scatter(ne, r3[h], zeros, _EH, _CHUNK_H,
                                    _NCHUNK_H))
        if i + 1 < len(blocks):
            wa, wb, _ = _edge_l1_split(blocks[i + 1])
        else:
            wa = params["dec"]["layers"][0][0]
            wb = wa
        nv = blk["node"]["layers"][0][0]
        nw = _mlp_params(blk["node"])
        nl, t_tab = _node_mlp_kernel(
            nl, aggs[0], aggs[1], nv[:LATENT], nv[LATENT:], nw[1], nw[2],
            nw[3], nw[4], nw[5], nw[6], nw[7], wa, wb)

    dec = params["dec"]["layers"]
    out = _decoder_kernel(t_tab, dec[0][1].reshape(1, -1),
                          dec[1][0], dec[1][1].reshape(1, -1),
                          dec[2][0], dec[2][1].reshape(1, -1))
    return out.reshape(1, N_NODES, -1)
